# Initial kernel scaffold; baseline (speedup 1.0000x reference)
import jax
import jax.numpy as jnp
from jax.experimental import pallas as pl


def kernel(feat, tar, real_mem, fake_mem, theta1_w, theta2_w):
    raise NotImplementedError("write your pallas kernel here")


# speedup vs baseline: 3.3249x; 3.3249x over previous
import functools

import jax
import jax.numpy as jnp
from jax import lax
from jax.experimental import pallas as pl
from jax.experimental.pallas import tpu as pltpu
from jax.experimental.pallas import tpu_sc as plsc

_N = 8192
_M = 2048
_D = 128
_R = 512
_NT = _N // _R
_RPT = 512
_L = 16


def _tc_a(feat_t, mems, interpret=False):
    def body(feat_ref, mems_ref, q_ref, rmr_ref, rmf_ref, gir_ref, gif_ref,
             cmr_ref, cmf_ref):
        t = pl.program_id(0)
        f = feat_ref[...]
        nrm = jnp.sqrt(jnp.sum(f * f, axis=1, keepdims=True))
        q = f / jnp.maximum(nrm, 1e-12)
        q_ref[...] = q
        iota = lax.broadcasted_iota(jnp.int32, (_R, _M), 1)
        outs = ((rmr_ref, gir_ref, cmr_ref), (rmf_ref, gif_ref, cmf_ref))
        for b in range(2):
            rm_ref, gi_ref, cm_ref = outs[b]
            mem = mems_ref[b]
            s = lax.dot_general(q, mem, (((1,), (1,)), ((), ())),
                                preferred_element_type=jnp.float32)
            rmax = jnp.max(s, axis=1)
            gi = jnp.min(jnp.where(s == rmax[:, None], iota, _M), axis=1)
            rm_ref[0, 0, :] = rmax
            gi_ref[0, 0, :] = gi
            cmt = jnp.max(s, axis=0)

            @pl.when(t == 0)
            def _():
                cm_ref[...] = cmt

            @pl.when(t != 0)
            def _():
                cm_ref[...] = jnp.maximum(cm_ref[...], cmt)

    f32 = jnp.float32
    return pl.pallas_call(
        body,
        grid=(_NT,),
        in_specs=[
            pl.BlockSpec((_R, _D), lambda t: (t, 0)),
            pl.BlockSpec((2, _M, _D), lambda t: (0, 0, 0)),
        ],
        out_specs=[
            pl.BlockSpec((_R, _D), lambda t: (t, 0)),
            pl.BlockSpec((1, 1, _R), lambda t: (t, 0, 0)),
            pl.BlockSpec((1, 1, _R), lambda t: (t, 0, 0)),
            pl.BlockSpec((1, 1, _R), lambda t: (t, 0, 0)),
            pl.BlockSpec((1, 1, _R), lambda t: (t, 0, 0)),
            pl.BlockSpec((_M,), lambda t: (0,)),
            pl.BlockSpec((_M,), lambda t: (0,)),
        ],
        out_shape=[
            jax.ShapeDtypeStruct((_N, _D), f32),
            jax.ShapeDtypeStruct((_NT, 1, _R), f32),
            jax.ShapeDtypeStruct((_NT, 1, _R), f32),
            jax.ShapeDtypeStruct((_NT, 1, _R), jnp.int32),
            jax.ShapeDtypeStruct((_NT, 1, _R), jnp.int32),
            jax.ShapeDtypeStruct((_M,), f32),
            jax.ShapeDtypeStruct((_M,), f32),
        ],
        interpret=interpret,
    )(feat_t, mems)


def _sc_b(q, rowmax, gidx, colmax):
    mesh = plsc.VectorSubcoreMesh(core_axis_name="c", subcore_axis_name="s")

    @functools.partial(
        pl.kernel,
        out_type=jax.ShapeDtypeStruct((2, _M, _D), jnp.float32),
        mesh=mesh,
        compiler_params=pltpu.CompilerParams(needs_layout_passes=False),
        scratch_types=[
            pltpu.VMEM((_RPT, _D), jnp.float32),
            pltpu.VMEM((_RPT,), jnp.float32),
            pltpu.VMEM((_RPT,), jnp.int32),
            pltpu.VMEM((_RPT // 128, 128), jnp.int32),
            pltpu.VMEM((_M,), jnp.float32),
            pltpu.VMEM((128, _D), jnp.float32),
            pltpu.VMEM_SHARED((_M, _D), jnp.float32),
        ],
    )
    def k(q_hbm, rm_hbm, gi_hbm, cm_hbm, qu_hbm,
          qv, rmv, giv, giv2, cmv, zv, shared):
        c = lax.axis_index("c")
        s = lax.axis_index("s")
        base = s * _RPT

        pltpu.sync_copy(q_hbm.at[pl.ds(base, _RPT)], qv)
        pltpu.sync_copy(rm_hbm.at[c, pl.ds(base, _RPT)], rmv)
        pltpu.sync_copy(gi_hbm.at[c, pl.ds(base, _RPT)], giv)
        for j in range(_RPT // 128):
            pltpu.sync_copy(gi_hbm.at[c, pl.ds(base + j * 128, 128)], giv2.at[j])
        pltpu.sync_copy(cm_hbm.at[c], cmv)

        zero16 = jnp.zeros((_L,), jnp.float32)

        def zloop(i, _):
            for k2 in range(_D // _L):
                zv[i, pl.ds(k2 * _L, _L)] = zero16
            return 0

        lax.fori_loop(0, 128, zloop, 0)
        pltpu.sync_copy(zv, shared.at[pl.ds(s * 128, 128)])

        def wsloop(i, _):
            off = i * _L
            g16 = giv[pl.ds(off, _L)]
            cm16 = plsc.load_gather(cmv, [g16])
            rm16 = rmv[pl.ds(off, _L)]
            w16 = jnp.exp(rm16 - cm16)
            for j in range(_L):
                wb = jnp.full((_L,), w16[j], jnp.float32)
                r = off + j
                for k2 in range(_D // _L):
                    qv[r, pl.ds(k2 * _L, _L)] = qv[r, pl.ds(k2 * _L, _L)] * wb
            return 0

        lax.fori_loop(0, _RPT // _L, wsloop, 0)

        plsc.subcore_barrier()
        for j in range(_RPT // 128):
            pltpu.sync_copy(qv.at[pl.ds(j * 128, 128)],
                            shared.at[giv2.at[j]], add=True)
        plsc.subcore_barrier()
        pltpu.sync_copy(shared.at[pl.ds(s * 128, 128)],
                        qu_hbm.at[c, pl.ds(s * 128, 128)])

    return k(q, rowmax, gidx, colmax)


def _tc_c(q, mems, qu, w1, w2, interpret=False):
    def body(q_ref, mems_ref, qu_ref, w1_ref, w2_ref, uq_ref, mem2_ref, m2s):
        t = pl.program_id(1)

        @pl.when(t == 0)
        def _():
            m = mems_ref[0] + qu_ref[0]
            nrm = jnp.sqrt(jnp.sum(m * m, axis=1, keepdims=True))
            m2 = m / jnp.maximum(nrm, 1e-12)
            m2s[...] = m2
            mem2_ref[0] = m2

        m2 = m2s[...]
        q = q_ref[...]
        s = lax.dot_general(q, m2, (((1,), (1,)), ((), ())),
                            preferred_element_type=jnp.float32)
        rmax = jnp.max(s, axis=1, keepdims=True)
        e = jnp.exp(s - rmax)
        den = jnp.sum(e, axis=1, keepdims=True)
        att2 = lax.dot_general(e, m2, (((1,), (0,)), ((), ())),
                               preferred_element_type=jnp.float32) / den
        f1 = lax.dot_general(q, w1_ref[...], (((1,), (1,)), ((), ())),
                             preferred_element_type=jnp.float32)
        f2 = lax.dot_general(att2, w2_ref[...], (((1,), (1,)), ((), ())),
                             preferred_element_type=jnp.float32)
        num = jnp.sum(f1 * f2, axis=1)
        den2 = jnp.sqrt(jnp.sum(f1 * f1, axis=1)) * jnp.sqrt(jnp.sum(f2 * f2, axis=1))
        sim = num / jnp.maximum(den2, 1e-8)
        uq_ref[0] = q * sim[:, None]

    f32 = jnp.float32
    return pl.pallas_call(
        body,
        grid=(2, _NT),
        in_specs=[
            pl.BlockSpec((_R, _D), lambda b, t: (t, 0)),
            pl.BlockSpec((1, _M, _D), lambda b, t: (b, 0, 0)),
            pl.BlockSpec((1, _M, _D), lambda b, t: (b, 0, 0)),
            pl.BlockSpec((_D, _D), lambda b, t: (0, 0)),
            pl.BlockSpec((_D, _D), lambda b, t: (0, 0)),
        ],
        out_specs=[
            pl.BlockSpec((1, _R, _D), lambda b, t: (b, t, 0)),
            pl.BlockSpec((1, _M, _D), lambda b, t: (b, 0, 0)),
        ],
        out_shape=[
            jax.ShapeDtypeStruct((2, _N, _D), f32),
            jax.ShapeDtypeStruct((2, _M, _D), f32),
        ],
        scratch_shapes=[pltpu.VMEM((_M, _D), f32)],
        interpret=interpret,
    )(q, mems, qu, w1, w2)


def kernel(feat, tar, real_mem, fake_mem, theta1_w, theta2_w):
    del tar
    b, d, h, w = feat.shape
    feat_t = feat.transpose(0, 2, 3, 1).reshape(_N, _D)
    mems = jnp.stack([real_mem, fake_mem])
    q, rmr, rmf, gir, gif, cmr, cmf = _tc_a(feat_t, mems)
    rowmax = jnp.stack([rmr.reshape(_N), rmf.reshape(_N)])
    gidx = jnp.stack([gir.reshape(_N), gif.reshape(_N)])
    colmax = jnp.stack([cmr, cmf])
    qu = _sc_b(q, rowmax, gidx, colmax)
    uq, mem2 = _tc_c(q, mems, qu, theta1_w, theta2_w)
    feat_out = q.reshape(b, h, w, d).transpose(0, 3, 1, 2)
    uq_r = uq[0].reshape(b, h, w, d).transpose(0, 3, 1, 2)
    uq_f = uq[1].reshape(b, h, w, d).transpose(0, 3, 1, 2)
    return uq_r, feat_out, mem2[0], uq_f, feat_out, mem2[1]


# speedup vs baseline: 3.6960x; 1.1116x over previous
import functools

import jax
import jax.numpy as jnp
from jax import lax
from jax.experimental import pallas as pl
from jax.experimental.pallas import tpu as pltpu
from jax.experimental.pallas import tpu_sc as plsc

_N = 8192
_M = 2048
_D = 128
_C = 512
_NS = 4096 // _C
_RPT = _N // 32
_L = 16


def _tc_a(feat_c, mems2):
    def body(feat_ref, mems_ref, qt_ref, qr_ref, rmr_ref, rmf_ref,
             gir_ref, gif_ref, cmr_ref, cmf_ref):
        b = pl.program_id(0)
        s = pl.program_id(1)
        f = feat_ref[0]
        nrm = jnp.sqrt(jnp.sum(f * f, axis=0, keepdims=True))
        q = f / jnp.maximum(nrm, 1e-12)
        qt_ref[0] = q
        qr_ref[...] = q.T
        st = lax.dot_general(mems_ref[...], q, (((1,), (0,)), ((), ())),
                             preferred_element_type=jnp.float32)
        iota = lax.broadcasted_iota(jnp.int32, (_M, _C), 0)
        outs = ((rmr_ref, gir_ref, cmr_ref), (rmf_ref, gif_ref, cmf_ref))
        for br in range(2):
            rm_ref, gi_ref, cm_ref = outs[br]
            sb = st[br * _M:(br + 1) * _M]
            rmax = jnp.max(sb, axis=0)
            gi = jnp.min(jnp.where(sb == rmax[None, :], iota, _M), axis=0)
            rm_ref[0, 0, :] = rmax
            gi_ref[0, 0, :] = gi
            cmt = jnp.max(sb, axis=1)

            @pl.when(jnp.logical_and(b == 0, s == 0))
            def _():
                cm_ref[...] = cmt

            @pl.when(jnp.logical_or(b != 0, s != 0))
            def _():
                cm_ref[...] = jnp.maximum(cm_ref[...], cmt)

    f32 = jnp.float32
    i32 = jnp.int32
    nt = 2 * _NS
    return pl.pallas_call(
        body,
        grid=(2, _NS),
        in_specs=[
            pl.BlockSpec((1, _D, _C), lambda b, s: (b, 0, s)),
            pl.BlockSpec((2 * _M, _D), lambda b, s: (0, 0)),
        ],
        out_specs=[
            pl.BlockSpec((1, _D, _C), lambda b, s: (b, 0, s)),
            pl.BlockSpec((_C, _D), lambda b, s: (b * _NS + s, 0)),
            pl.BlockSpec((1, 1, _C), lambda b, s: (b * _NS + s, 0, 0)),
            pl.BlockSpec((1, 1, _C), lambda b, s: (b * _NS + s, 0, 0)),
            pl.BlockSpec((1, 1, _C), lambda b, s: (b * _NS + s, 0, 0)),
            pl.BlockSpec((1, 1, _C), lambda b, s: (b * _NS + s, 0, 0)),
            pl.BlockSpec((_M,), lambda b, s: (0,)),
            pl.BlockSpec((_M,), lambda b, s: (0,)),
        ],
        out_shape=[
            jax.ShapeDtypeStruct((2, _D, 4096), f32),
            jax.ShapeDtypeStruct((_N, _D), f32),
            jax.ShapeDtypeStruct((nt, 1, _C), f32),
            jax.ShapeDtypeStruct((nt, 1, _C), f32),
            jax.ShapeDtypeStruct((nt, 1, _C), i32),
            jax.ShapeDtypeStruct((nt, 1, _C), i32),
            jax.ShapeDtypeStruct((_M,), f32),
            jax.ShapeDtypeStruct((_M,), f32),
        ],
    )(feat_c, mems2)


def _sc_b(q_rows, rowmax, gidx, colmax):
    mesh = plsc.VectorSubcoreMesh(core_axis_name="c", subcore_axis_name="s")

    @functools.partial(
        pl.kernel,
        out_type=jax.ShapeDtypeStruct((2, _M, _D), jnp.float32),
        mesh=mesh,
        compiler_params=pltpu.CompilerParams(needs_layout_passes=False),
        scratch_types=[
            pltpu.VMEM((_RPT, _D), jnp.float32),
            pltpu.VMEM((_RPT,), jnp.float32),
            pltpu.VMEM((_RPT,), jnp.int32),
            pltpu.VMEM((_RPT // 128, 128), jnp.int32),
            pltpu.VMEM((_M,), jnp.float32),
            pltpu.VMEM((_M // 16, _D), jnp.float32),
            pltpu.VMEM_SHARED((_M, _D), jnp.float32),
        ],
    )
    def k(q_hbm, rm_hbm, gi_hbm, cm_hbm, qu_hbm,
          qv, rmv, giv, giv2, cmv, zv, shared):
        c = lax.axis_index("c")
        s = lax.axis_index("s")
        wid = s * 2 + c
        base = wid * _RPT

        pltpu.sync_copy(q_hbm.at[pl.ds(base, _RPT)], qv)
        pltpu.sync_copy(rm_hbm.at[pl.ds(base, _RPT)], rmv)
        pltpu.sync_copy(gi_hbm.at[pl.ds(base, _RPT)], giv)
        for j in range(_RPT // 128):
            pltpu.sync_copy(gi_hbm.at[pl.ds(base + j * 128, 128)], giv2.at[j])
        pltpu.sync_copy(cm_hbm, cmv)

        zero16 = jnp.zeros((_L,), jnp.float32)
        zrows = _M // 16

        def zloop(i, _):
            for k2 in range(_D // _L):
                zv[i, pl.ds(k2 * _L, _L)] = zero16
            return 0

        lax.fori_loop(0, zrows, zloop, 0)
        pltpu.sync_copy(zv, shared.at[pl.ds(s * zrows, zrows)])

        def wsloop(i, _):
            off = i * _L
            g16 = giv[pl.ds(off, _L)]
            cm16 = plsc.load_gather(cmv, [g16])
            rm16 = rmv[pl.ds(off, _L)]
            w16 = jnp.exp(rm16 - cm16)
            for j in range(_L):
                wb = jnp.full((_L,), w16[j], jnp.float32)
                r = off + j
                for k2 in range(_D // _L):
                    qv[r, pl.ds(k2 * _L, _L)] = qv[r, pl.ds(k2 * _L, _L)] * wb
            return 0

        lax.fori_loop(0, _RPT // _L, wsloop, 0)

        plsc.subcore_barrier()
        for j in range(_RPT // 128):
            pltpu.sync_copy(qv.at[pl.ds(j * 128, 128)],
                            shared.at[giv2.at[j]], add=True)
        plsc.subcore_barrier()
        pltpu.sync_copy(shared.at[pl.ds(s * zrows, zrows)],
                        qu_hbm.at[c, pl.ds(s * zrows, zrows)])

    return k(q_rows, rowmax, gidx, colmax)


def _tc_c(q_t, mem, qu):
    def body(q_ref, mem_ref, qu_ref, w1_ref, w2_ref, uq_ref, mem2_ref, m2s):
        b = pl.program_id(0)
        s = pl.program_id(1)

        @pl.when(jnp.logical_and(b == 0, s == 0))
        def _():
            m = mem_ref[...] + qu_ref[0] + qu_ref[1]
            nrm = jnp.sqrt(jnp.sum(m * m, axis=1, keepdims=True))
            m2 = m / jnp.maximum(nrm, 1e-12)
            m2s[...] = m2
            mem2_ref[...] = m2

        m2 = m2s[...]
        q = q_ref[0]
        st = lax.dot_general(m2, q, (((1,), (0,)), ((), ())),
                             preferred_element_type=jnp.float32)
        rmax = jnp.max(st, axis=0, keepdims=True)
        e = jnp.exp(st - rmax)
        den = jnp.sum(e, axis=0, keepdims=True)
        att2 = lax.dot_general(m2, e, (((0,), (0,)), ((), ())),
                               preferred_element_type=jnp.float32)
        att2 = att2 / den
        f1 = lax.dot_general(w1_ref[...], q, (((1,), (0,)), ((), ())),
                             preferred_element_type=jnp.float32)
        f2 = lax.dot_general(w2_ref[...], att2, (((1,), (0,)), ((), ())),
                             preferred_element_type=jnp.float32)
        num = jnp.sum(f1 * f2, axis=0)
        den2 = jnp.sqrt(jnp.sum(f1 * f1, axis=0)) * jnp.sqrt(jnp.sum(f2 * f2, axis=0))
        sim = num / jnp.maximum(den2, 1e-8)
        uq_ref[0] = q * sim[None, :]

    f32 = jnp.float32

    def call(w1, w2):
        return pl.pallas_call(
            body,
            grid=(2, _NS),
            in_specs=[
                pl.BlockSpec((1, _D, _C), lambda b, s: (b, 0, s)),
                pl.BlockSpec((_M, _D), lambda b, s: (0, 0)),
                pl.BlockSpec((2, _M, _D), lambda b, s: (0, 0, 0)),
                pl.BlockSpec((_D, _D), lambda b, s: (0, 0)),
                pl.BlockSpec((_D, _D), lambda b, s: (0, 0)),
            ],
            out_specs=[
                pl.BlockSpec((1, _D, _C), lambda b, s: (b, 0, s)),
                pl.BlockSpec((_M, _D), lambda b, s: (0, 0)),
            ],
            out_shape=[
                jax.ShapeDtypeStruct((2, _D, 4096), f32),
                jax.ShapeDtypeStruct((_M, _D), f32),
            ],
            scratch_shapes=[pltpu.VMEM((_M, _D), f32)],
        )(q_t, mem, qu, w1, w2)

    return call


def kernel(feat, tar, real_mem, fake_mem, theta1_w, theta2_w):
    del tar
    b, d, h, w = feat.shape
    feat_c = feat.reshape(b, d, h * w)
    mems2 = jnp.concatenate([real_mem, fake_mem], axis=0)
    (q_t, q_rows, rmr, rmf, gir, gif, cmr, cmf) = _tc_a(feat_c, mems2)
    qu_r = _sc_b(q_rows, rmr.reshape(_N), gir.reshape(_N), cmr)
    qu_f = _sc_b(q_rows, rmf.reshape(_N), gif.reshape(_N), cmf)
    uq_r, mem2_r = _tc_c(q_t, real_mem, qu_r)(theta1_w, theta2_w)
    uq_f, mem2_f = _tc_c(q_t, fake_mem, qu_f)(theta1_w, theta2_w)
    feat_out = q_t.reshape(b, d, h, w)
    return (uq_r.reshape(b, d, h, w), feat_out, mem2_r,
            uq_f.reshape(b, d, h, w), feat_out, mem2_f)


# speedup vs baseline: 4.1523x; 1.1235x over previous
import functools

import jax
import jax.numpy as jnp
from jax import lax
from jax.experimental import pallas as pl
from jax.experimental.pallas import tpu as pltpu
from jax.experimental.pallas import tpu_sc as plsc

def _eye(n):
    r = lax.broadcasted_iota(jnp.int32, (n, n), 0)
    c = lax.broadcasted_iota(jnp.int32, (n, n), 1)
    return (r == c).astype(jnp.float32)


_N = 8192
_M = 2048
_D = 128
_C = 512
_NS = 4096 // _C
_RPT = _N // 16
_L = 16


def _tc_a(feat_c, mems2t):
    def body(feat_ref, mems_ref, qt_ref, qr_ref, rmr_ref, rmf_ref,
             gir_ref, gif_ref, cmr_ref, cmf_ref):
        b = pl.program_id(0)
        s = pl.program_id(1)
        f = feat_ref[0]
        nrm = jnp.sqrt(jnp.sum(f * f, axis=0, keepdims=True))
        q = f / jnp.maximum(nrm, 1e-12)
        qt_ref[0] = q
        qr = lax.dot_general(q, _eye(_D), (((0,), (0,)), ((), ())),
                             preferred_element_type=jnp.float32)
        qr_ref[...] = qr
        st = lax.dot_general(qr, mems_ref[...], (((1,), (0,)), ((), ())),
                             preferred_element_type=jnp.float32)
        outs = ((rmr_ref, gir_ref, cmr_ref), (rmf_ref, gif_ref, cmf_ref))
        for br in range(2):
            rm_ref, gi_ref, cm_ref = outs[br]
            sb = st[:, br * _M:(br + 1) * _M]
            rmax = jnp.max(sb, axis=1)
            gi = jnp.argmax(sb, axis=1).astype(jnp.int32)
            rm_ref[0, :, 0] = rmax
            gi_ref[0, :, 0] = gi
            cmt = jnp.max(sb, axis=0)

            @pl.when(jnp.logical_and(b == 0, s == 0))
            def _():
                cm_ref[...] = cmt

            @pl.when(jnp.logical_or(b != 0, s != 0))
            def _():
                cm_ref[...] = jnp.maximum(cm_ref[...], cmt)

    f32 = jnp.float32
    i32 = jnp.int32
    nt = 2 * _NS
    return pl.pallas_call(
        body,
        grid=(2, _NS),
        in_specs=[
            pl.BlockSpec((1, _D, _C), lambda b, s: (b, 0, s)),
            pl.BlockSpec((_D, 2 * _M), lambda b, s: (0, 0)),
        ],
        out_specs=[
            pl.BlockSpec((1, _D, _C), lambda b, s: (b, 0, s)),
            pl.BlockSpec((_C, _D), lambda b, s: (b * _NS + s, 0)),
            pl.BlockSpec((1, _C, 1), lambda b, s: (b * _NS + s, 0, 0)),
            pl.BlockSpec((1, _C, 1), lambda b, s: (b * _NS + s, 0, 0)),
            pl.BlockSpec((1, _C, 1), lambda b, s: (b * _NS + s, 0, 0)),
            pl.BlockSpec((1, _C, 1), lambda b, s: (b * _NS + s, 0, 0)),
            pl.BlockSpec((_M,), lambda b, s: (0,)),
            pl.BlockSpec((_M,), lambda b, s: (0,)),
        ],
        out_shape=[
            jax.ShapeDtypeStruct((2, _D, 4096), f32),
            jax.ShapeDtypeStruct((_N, _D), f32),
            jax.ShapeDtypeStruct((nt, _C, 1), f32),
            jax.ShapeDtypeStruct((nt, _C, 1), f32),
            jax.ShapeDtypeStruct((nt, _C, 1), i32),
            jax.ShapeDtypeStruct((nt, _C, 1), i32),
            jax.ShapeDtypeStruct((_M,), f32),
            jax.ShapeDtypeStruct((_M,), f32),
        ],
    )(feat_c, mems2t)


def _sc_b(q_rows, rowmax, gidx, colmax):
    mesh = plsc.VectorSubcoreMesh(core_axis_name="c", subcore_axis_name="s")

    @functools.partial(
        pl.kernel,
        out_type=jax.ShapeDtypeStruct((2, _M, _D), jnp.float32),
        mesh=mesh,
        compiler_params=pltpu.CompilerParams(needs_layout_passes=False),
        scratch_types=[
            pltpu.VMEM((_RPT, _D), jnp.float32),
            pltpu.VMEM((_RPT,), jnp.float32),
            pltpu.VMEM((_RPT,), jnp.int32),
            pltpu.VMEM((_RPT // 128, 128), jnp.int32),
            pltpu.VMEM((_M,), jnp.float32),
            pltpu.VMEM((_M // 16, _D), jnp.float32),
            pltpu.VMEM_SHARED((_M, _D), jnp.float32),
        ],
    )
    def k(q_hbm, rm_hbm, gi_hbm, cm_hbm, qu_hbm,
          qv, rmv, giv, giv2, cmv, zv, shared):
        c = lax.axis_index("c")
        s = lax.axis_index("s")
        base = s * _RPT

        pltpu.sync_copy(q_hbm.at[pl.ds(base, _RPT)], qv)
        pltpu.sync_copy(rm_hbm.at[c, pl.ds(base, _RPT)], rmv)
        pltpu.sync_copy(gi_hbm.at[c, pl.ds(base, _RPT)], giv)
        for j in range(_RPT // 128):
            pltpu.sync_copy(gi_hbm.at[c, pl.ds(base + j * 128, 128)], giv2.at[j])
        pltpu.sync_copy(cm_hbm.at[c], cmv)

        zero16 = jnp.zeros((_L,), jnp.float32)
        zrows = _M // 16

        def zloop(i, _):
            for k2 in range(_D // _L):
                zv[i, pl.ds(k2 * _L, _L)] = zero16
            return 0

        lax.fori_loop(0, zrows, zloop, 0)
        pltpu.sync_copy(zv, shared.at[pl.ds(s * zrows, zrows)])

        def wsloop(i, _):
            off = i * _L
            g16 = giv[pl.ds(off, _L)]
            cm16 = plsc.load_gather(cmv, [g16])
            rm16 = rmv[pl.ds(off, _L)]
            w16 = jnp.exp(rm16 - cm16)
            for j in range(_L):
                wb = jnp.full((_L,), w16[j], jnp.float32)
                r = off + j
                for k2 in range(_D // _L):
                    qv[r, pl.ds(k2 * _L, _L)] = qv[r, pl.ds(k2 * _L, _L)] * wb
            return 0

        lax.fori_loop(0, _RPT // _L, wsloop, 0)

        plsc.subcore_barrier()
        for j in range(_RPT // 128):
            pltpu.sync_copy(qv.at[pl.ds(j * 128, 128)],
                            shared.at[giv2.at[j]], add=True)
        plsc.subcore_barrier()
        pltpu.sync_copy(shared.at[pl.ds(s * zrows, zrows)],
                        qu_hbm.at[c, pl.ds(s * zrows, zrows)])

    return k(q_rows, rowmax, gidx, colmax)


def _tc_c(q_t, mem, qu, br):
    def body(q_ref, mem_ref, qu_ref, w1_ref, w2_ref, uq_ref, mem2_ref,
             m2s, m2ts):
        b = pl.program_id(0)
        s = pl.program_id(1)

        @pl.when(jnp.logical_and(b == 0, s == 0))
        def _():
            m = mem_ref[...] + qu_ref[0]
            nrm = jnp.sqrt(jnp.sum(m * m, axis=1, keepdims=True))
            m2 = m / jnp.maximum(nrm, 1e-12)
            m2s[...] = m2
            mem2_ref[...] = m2
            m2ts[...] = lax.dot_general(_eye(_D), m2, (((1,), (1,)), ((), ())),
                                        preferred_element_type=jnp.float32)

        m2 = m2s[...]
        q = q_ref[0]
        st = lax.dot_general(m2, q, (((1,), (0,)), ((), ())),
                             preferred_element_type=jnp.float32)
        e = jnp.exp(st)
        den = jnp.sum(e, axis=0, keepdims=True)
        att2 = lax.dot_general(m2ts[...], e, (((1,), (0,)), ((), ())),
                               preferred_element_type=jnp.float32)
        att2 = att2 / den
        f1 = lax.dot_general(w1_ref[...], q, (((1,), (0,)), ((), ())),
                             preferred_element_type=jnp.float32)
        f2 = lax.dot_general(w2_ref[...], att2, (((1,), (0,)), ((), ())),
                             preferred_element_type=jnp.float32)
        num = jnp.sum(f1 * f2, axis=0)
        den2 = jnp.sqrt(jnp.sum(f1 * f1, axis=0)) * jnp.sqrt(jnp.sum(f2 * f2, axis=0))
        sim = num / jnp.maximum(den2, 1e-8)
        uq_ref[0] = q * sim[None, :]

    f32 = jnp.float32

    def call(w1, w2):
        return pl.pallas_call(
            body,
            grid=(2, _NS),
            in_specs=[
                pl.BlockSpec((1, _D, _C), lambda b, s: (b, 0, s)),
                pl.BlockSpec((_M, _D), lambda b, s: (0, 0)),
                pl.BlockSpec((1, _M, _D), lambda b, s: (br, 0, 0)),
                pl.BlockSpec((_D, _D), lambda b, s: (0, 0)),
                pl.BlockSpec((_D, _D), lambda b, s: (0, 0)),
            ],
            out_specs=[
                pl.BlockSpec((1, _D, _C), lambda b, s: (b, 0, s)),
                pl.BlockSpec((_M, _D), lambda b, s: (0, 0)),
            ],
            out_shape=[
                jax.ShapeDtypeStruct((2, _D, 4096), f32),
                jax.ShapeDtypeStruct((_M, _D), f32),
            ],
            scratch_shapes=[pltpu.VMEM((_M, _D), f32),
                            pltpu.VMEM((_D, _M), f32)],
        )(q_t, mem, qu, w1, w2)

    return call


def kernel(feat, tar, real_mem, fake_mem, theta1_w, theta2_w):
    del tar
    b, d, h, w = feat.shape
    feat_c = feat.reshape(b, d, h * w)
    mems2t = jnp.concatenate([real_mem, fake_mem], axis=0).T
    (q_t, q_rows, rmr, rmf, gir, gif, cmr, cmf) = _tc_a(feat_c, mems2t)
    rowmax = jnp.stack([rmr.reshape(_N), rmf.reshape(_N)])
    gidx = jnp.stack([gir.reshape(_N), gif.reshape(_N)])
    colmax = jnp.stack([cmr, cmf])
    qu = _sc_b(q_rows, rowmax, gidx, colmax)
    uq_r, mem2_r = _tc_c(q_t, real_mem, qu, 0)(theta1_w, theta2_w)
    uq_f, mem2_f = _tc_c(q_t, fake_mem, qu, 1)(theta1_w, theta2_w)
    feat_out = q_t.reshape(b, d, h, w)
    return (uq_r.reshape(b, d, h, w), feat_out, mem2_r,
            uq_f.reshape(b, d, h, w), feat_out, mem2_f)


# speedup vs baseline: 4.2503x; 1.0236x over previous
import functools

import jax
import jax.numpy as jnp
from jax import lax
from jax.experimental import pallas as pl
from jax.experimental.pallas import tpu as pltpu
from jax.experimental.pallas import tpu_sc as plsc

def _eye(n):
    r = lax.broadcasted_iota(jnp.int32, (n, n), 0)
    c = lax.broadcasted_iota(jnp.int32, (n, n), 1)
    return (r == c).astype(jnp.float32)


_N = 8192
_M = 2048
_D = 128
_C = 512
_NS = 4096 // _C
_RPT = _N // 16
_L = 16


def _tc_a(feat_c, mems2t):
    def body(feat_ref, mems_ref, qt_ref, qr_ref, rmr_ref, rmf_ref,
             gir_ref, gif_ref, cmr_ref, cmf_ref):
        b = pl.program_id(0)
        s = pl.program_id(1)
        f = feat_ref[0]
        nrm = jnp.sqrt(jnp.sum(f * f, axis=0, keepdims=True))
        q = f / jnp.maximum(nrm, 1e-12)
        qt_ref[0] = q
        qr = lax.dot_general(q, _eye(_D), (((0,), (0,)), ((), ())),
                             preferred_element_type=jnp.float32)
        qr_ref[...] = qr
        st = lax.dot_general(qr, mems_ref[...], (((1,), (0,)), ((), ())),
                             preferred_element_type=jnp.float32)
        outs = ((rmr_ref, gir_ref, cmr_ref), (rmf_ref, gif_ref, cmf_ref))
        for br in range(2):
            rm_ref, gi_ref, cm_ref = outs[br]
            sb = st[:, br * _M:(br + 1) * _M]
            rmax = jnp.max(sb, axis=1)
            gi = jnp.argmax(sb, axis=1).astype(jnp.int32)
            rm_ref[0, :, 0] = rmax
            gi_ref[0, :, 0] = gi
            cmt = jnp.max(sb, axis=0)

            @pl.when(jnp.logical_and(b == 0, s == 0))
            def _():
                cm_ref[...] = cmt

            @pl.when(jnp.logical_or(b != 0, s != 0))
            def _():
                cm_ref[...] = jnp.maximum(cm_ref[...], cmt)

    f32 = jnp.float32
    i32 = jnp.int32
    nt = 2 * _NS
    return pl.pallas_call(
        body,
        grid=(2, _NS),
        in_specs=[
            pl.BlockSpec((1, _D, _C), lambda b, s: (b, 0, s)),
            pl.BlockSpec((_D, 2 * _M), lambda b, s: (0, 0)),
        ],
        out_specs=[
            pl.BlockSpec((1, _D, _C), lambda b, s: (b, 0, s)),
            pl.BlockSpec((_C, _D), lambda b, s: (b * _NS + s, 0)),
            pl.BlockSpec((1, _C, 1), lambda b, s: (b * _NS + s, 0, 0)),
            pl.BlockSpec((1, _C, 1), lambda b, s: (b * _NS + s, 0, 0)),
            pl.BlockSpec((1, _C, 1), lambda b, s: (b * _NS + s, 0, 0)),
            pl.BlockSpec((1, _C, 1), lambda b, s: (b * _NS + s, 0, 0)),
            pl.BlockSpec((_M,), lambda b, s: (0,)),
            pl.BlockSpec((_M,), lambda b, s: (0,)),
        ],
        out_shape=[
            jax.ShapeDtypeStruct((2, _D, 4096), f32),
            jax.ShapeDtypeStruct((_N, _D), f32),
            jax.ShapeDtypeStruct((nt, _C, 1), f32),
            jax.ShapeDtypeStruct((nt, _C, 1), f32),
            jax.ShapeDtypeStruct((nt, _C, 1), i32),
            jax.ShapeDtypeStruct((nt, _C, 1), i32),
            jax.ShapeDtypeStruct((_M,), f32),
            jax.ShapeDtypeStruct((_M,), f32),
        ],
    )(feat_c, mems2t)


def _sc_b(q_rows, rowmax, gidx, colmax):
    mesh = plsc.VectorSubcoreMesh(core_axis_name="c", subcore_axis_name="s")

    @functools.partial(
        pl.kernel,
        out_type=jax.ShapeDtypeStruct((2, _M, _D), jnp.float32),
        mesh=mesh,
        compiler_params=pltpu.CompilerParams(needs_layout_passes=False),
        scratch_types=[
            pltpu.VMEM((_RPT, _D), jnp.float32),
            pltpu.VMEM((_RPT,), jnp.float32),
            pltpu.VMEM((_RPT,), jnp.int32),
            pltpu.VMEM((_RPT // 128, 128), jnp.int32),
            pltpu.VMEM((_M,), jnp.float32),
            pltpu.VMEM((_M // 16, _D), jnp.float32),
            pltpu.VMEM_SHARED((_M, _D), jnp.float32),
            pltpu.SemaphoreType.DMA,
        ],
    )
    def k(q_hbm, rm_hbm, gi_hbm, cm_hbm, qu_hbm,
          qv, rmv, giv, giv2, cmv, zv, shared, sem):
        c = lax.axis_index("c")
        s = lax.axis_index("s")
        base = s * _RPT

        copies = [
            pltpu.async_copy(q_hbm.at[pl.ds(base, _RPT)], qv, sem),
            pltpu.async_copy(rm_hbm.at[c, pl.ds(base, _RPT)], rmv, sem),
            pltpu.async_copy(gi_hbm.at[c, pl.ds(base, _RPT)], giv, sem),
            pltpu.async_copy(cm_hbm.at[c], cmv, sem),
        ] + [
            pltpu.async_copy(gi_hbm.at[c, pl.ds(base + j * 128, 128)],
                             giv2.at[j], sem)
            for j in range(_RPT // 128)
        ]
        zero16 = jnp.zeros((_L,), jnp.float32)
        zrows = _M // 16

        def zloop(i, _):
            for k2 in range(_D // _L):
                zv[i, pl.ds(k2 * _L, _L)] = zero16
            return 0

        lax.fori_loop(0, zrows, zloop, 0)
        pltpu.sync_copy(zv, shared.at[pl.ds(s * zrows, zrows)])

        for cp in copies:
            cp.wait()

        def wsloop(i, _):
            off = i * _L
            g16 = giv[pl.ds(off, _L)]
            cm16 = plsc.load_gather(cmv, [g16])
            rm16 = rmv[pl.ds(off, _L)]
            w16 = jnp.exp(rm16 - cm16)
            for j in range(_L):
                wb = jnp.full((_L,), w16[j], jnp.float32)
                r = off + j
                for k2 in range(_D // _L):
                    qv[r, pl.ds(k2 * _L, _L)] = qv[r, pl.ds(k2 * _L, _L)] * wb
            return 0

        lax.fori_loop(0, _RPT // _L, wsloop, 0)

        plsc.subcore_barrier()
        scats = [
            pltpu.async_copy(qv.at[pl.ds(j * 128, 128)],
                             shared.at[giv2.at[j]], sem, add=True)
            for j in range(_RPT // 128)
        ]
        for cp in scats:
            cp.wait()
        plsc.subcore_barrier()
        pltpu.sync_copy(shared.at[pl.ds(s * zrows, zrows)],
                        qu_hbm.at[c, pl.ds(s * zrows, zrows)])

    return k(q_rows, rowmax, gidx, colmax)


def _tc_c(q_t, mem, qu, br):
    def body(q_ref, mem_ref, qu_ref, w1_ref, w2_ref, uq_ref, mem2_ref,
             m2s, m2ts):
        b = pl.program_id(0)
        s = pl.program_id(1)

        @pl.when(jnp.logical_and(b == 0, s == 0))
        def _():
            m = mem_ref[...] + qu_ref[0]
            nrm = jnp.sqrt(jnp.sum(m * m, axis=1, keepdims=True))
            m2 = m / jnp.maximum(nrm, 1e-12)
            m2s[...] = m2
            mem2_ref[...] = m2
            m2ts[...] = lax.dot_general(_eye(_D), m2, (((1,), (1,)), ((), ())),
                                        preferred_element_type=jnp.float32)

        m2 = m2s[...]
        q = q_ref[0]
        st = lax.dot_general(m2, q, (((1,), (0,)), ((), ())),
                             preferred_element_type=jnp.float32)
        e = jnp.exp(st)
        den = jnp.sum(e, axis=0, keepdims=True)
        att2 = lax.dot_general(m2ts[...], e, (((1,), (0,)), ((), ())),
                               preferred_element_type=jnp.float32)
        att2 = att2 / den
        f1 = lax.dot_general(w1_ref[...], q, (((1,), (0,)), ((), ())),
                             preferred_element_type=jnp.float32)
        f2 = lax.dot_general(w2_ref[...], att2, (((1,), (0,)), ((), ())),
                             preferred_element_type=jnp.float32)
        num = jnp.sum(f1 * f2, axis=0)
        den2 = jnp.sqrt(jnp.sum(f1 * f1, axis=0)) * jnp.sqrt(jnp.sum(f2 * f2, axis=0))
        sim = num / jnp.maximum(den2, 1e-8)
        uq_ref[0] = q * sim[None, :]

    f32 = jnp.float32

    def call(w1, w2):
        return pl.pallas_call(
            body,
            grid=(2, _NS),
            in_specs=[
                pl.BlockSpec((1, _D, _C), lambda b, s: (b, 0, s)),
                pl.BlockSpec((_M, _D), lambda b, s: (0, 0)),
                pl.BlockSpec((1, _M, _D), lambda b, s: (br, 0, 0)),
                pl.BlockSpec((_D, _D), lambda b, s: (0, 0)),
                pl.BlockSpec((_D, _D), lambda b, s: (0, 0)),
            ],
            out_specs=[
                pl.BlockSpec((1, _D, _C), lambda b, s: (b, 0, s)),
                pl.BlockSpec((_M, _D), lambda b, s: (0, 0)),
            ],
            out_shape=[
                jax.ShapeDtypeStruct((2, _D, 4096), f32),
                jax.ShapeDtypeStruct((_M, _D), f32),
            ],
            scratch_shapes=[pltpu.VMEM((_M, _D), f32),
                            pltpu.VMEM((_D, _M), f32)],
        )(q_t, mem, qu, w1, w2)

    return call


def kernel(feat, tar, real_mem, fake_mem, theta1_w, theta2_w):
    del tar
    b, d, h, w = feat.shape
    feat_c = feat.reshape(b, d, h * w)
    mems2t = jnp.concatenate([real_mem, fake_mem], axis=0).T
    (q_t, q_rows, rmr, rmf, gir, gif, cmr, cmf) = _tc_a(feat_c, mems2t)
    rowmax = jnp.stack([rmr.reshape(_N), rmf.reshape(_N)])
    gidx = jnp.stack([gir.reshape(_N), gif.reshape(_N)])
    colmax = jnp.stack([cmr, cmf])
    qu = _sc_b(q_rows, rowmax, gidx, colmax)
    uq_r, mem2_r = _tc_c(q_t, real_mem, qu, 0)(theta1_w, theta2_w)
    uq_f, mem2_f = _tc_c(q_t, fake_mem, qu, 1)(theta1_w, theta2_w)
    feat_out = q_t.reshape(b, d, h, w)
    return (uq_r.reshape(b, d, h, w), feat_out, mem2_r,
            uq_f.reshape(b, d, h, w), feat_out, mem2_f)


# speedup vs baseline: 4.3500x; 1.0234x over previous
import functools

import jax
import jax.numpy as jnp
from jax import lax
from jax.experimental import pallas as pl
from jax.experimental.pallas import tpu as pltpu
from jax.experimental.pallas import tpu_sc as plsc

def _eye(n):
    r = lax.broadcasted_iota(jnp.int32, (n, n), 0)
    c = lax.broadcasted_iota(jnp.int32, (n, n), 1)
    return (r == c).astype(jnp.float32)


_N = 8192
_M = 2048
_D = 128
_C = 512
_NS = 4096 // _C
_RPT = _N // 16
_L = 16


def _tc_a(feat_c, real_mem, fake_mem):
    def body(feat_ref, rmem_ref, fmem_ref, qt_ref, qr_ref, rm_ref,
             gi_ref, cm_ref, mt):
        b = pl.program_id(0)
        s = pl.program_id(1)

        @pl.when(jnp.logical_and(b == 0, s == 0))
        def _():
            eye = _eye(_D)
            mt[:, :_M] = lax.dot_general(eye, rmem_ref[...],
                                         (((1,), (1,)), ((), ())),
                                         preferred_element_type=jnp.float32)
            mt[:, _M:] = lax.dot_general(eye, fmem_ref[...],
                                         (((1,), (1,)), ((), ())),
                                         preferred_element_type=jnp.float32)

        f = feat_ref[0]
        nrm = jnp.sqrt(jnp.sum(f * f, axis=0, keepdims=True))
        q = f / jnp.maximum(nrm, 1e-12)
        qt_ref[0] = q
        qr = lax.dot_general(q, _eye(_D), (((0,), (0,)), ((), ())),
                             preferred_element_type=jnp.float32)
        qr_ref[...] = qr
        st = lax.dot_general(qr, mt[...], (((1,), (0,)), ((), ())),
                             preferred_element_type=jnp.float32)
        for br in range(2):
            sb = st[:, br * _M:(br + 1) * _M]
            rmax = jnp.max(sb, axis=1)
            gi = jnp.argmax(sb, axis=1).astype(jnp.int32)
            rm_ref[br, 0, :, 0] = rmax
            gi_ref[br, 0, :, 0] = gi
            cmt = jnp.max(sb, axis=0)

            @pl.when(jnp.logical_and(b == 0, s == 0))
            def _():
                cm_ref[br, 0, :] = cmt

            @pl.when(jnp.logical_or(b != 0, s != 0))
            def _():
                cm_ref[br, 0, :] = jnp.maximum(cm_ref[br, 0, :], cmt)

    f32 = jnp.float32
    i32 = jnp.int32
    nt = 2 * _NS
    return pl.pallas_call(
        body,
        grid=(2, _NS),
        in_specs=[
            pl.BlockSpec((1, _D, _C), lambda b, s: (b, 0, s)),
            pl.BlockSpec((_M, _D), lambda b, s: (0, 0)),
            pl.BlockSpec((_M, _D), lambda b, s: (0, 0)),
        ],
        out_specs=[
            pl.BlockSpec((1, _D, _C), lambda b, s: (b, 0, s)),
            pl.BlockSpec((_C, _D), lambda b, s: (b * _NS + s, 0)),
            pl.BlockSpec((2, 1, _C, 1), lambda b, s: (0, b * _NS + s, 0, 0)),
            pl.BlockSpec((2, 1, _C, 1), lambda b, s: (0, b * _NS + s, 0, 0)),
            pl.BlockSpec((2, 1, _M), lambda b, s: (0, 0, 0)),
        ],
        out_shape=[
            jax.ShapeDtypeStruct((2, _D, 4096), f32),
            jax.ShapeDtypeStruct((_N, _D), f32),
            jax.ShapeDtypeStruct((2, nt, _C, 1), f32),
            jax.ShapeDtypeStruct((2, nt, _C, 1), i32),
            jax.ShapeDtypeStruct((2, 1, _M), f32),
        ],
        scratch_shapes=[pltpu.VMEM((_D, 2 * _M), f32)],
    )(feat_c, real_mem, fake_mem)


def _sc_b(q_rows, rowmax, gidx, colmax):
    mesh = plsc.VectorSubcoreMesh(core_axis_name="c", subcore_axis_name="s")

    @functools.partial(
        pl.kernel,
        out_type=jax.ShapeDtypeStruct((2, _M, _D), jnp.float32),
        mesh=mesh,
        compiler_params=pltpu.CompilerParams(needs_layout_passes=False),
        scratch_types=[
            pltpu.VMEM((_RPT, _D), jnp.float32),
            pltpu.VMEM((_RPT,), jnp.float32),
            pltpu.VMEM((_RPT,), jnp.int32),
            pltpu.VMEM((_RPT // 128, 128), jnp.int32),
            pltpu.VMEM((_M,), jnp.float32),
            pltpu.VMEM((_M // 16, _D), jnp.float32),
            pltpu.VMEM_SHARED((_M, _D), jnp.float32),
            pltpu.SemaphoreType.DMA,
        ],
    )
    def k(q_hbm, rm_hbm, gi_hbm, cm_hbm, qu_hbm,
          qv, rmv, giv, giv2, cmv, zv, shared, sem):
        c = lax.axis_index("c")
        s = lax.axis_index("s")
        base = s * _RPT

        copies = [
            pltpu.async_copy(q_hbm.at[pl.ds(base, _RPT)], qv, sem),
            pltpu.async_copy(rm_hbm.at[c, pl.ds(base, _RPT)], rmv, sem),
            pltpu.async_copy(gi_hbm.at[c, pl.ds(base, _RPT)], giv, sem),
            pltpu.async_copy(cm_hbm.at[c], cmv, sem),
        ] + [
            pltpu.async_copy(gi_hbm.at[c, pl.ds(base + j * 128, 128)],
                             giv2.at[j], sem)
            for j in range(_RPT // 128)
        ]
        zero16 = jnp.zeros((_L,), jnp.float32)
        zrows = _M // 16

        def zloop(i, _):
            for k2 in range(_D // _L):
                zv[i, pl.ds(k2 * _L, _L)] = zero16
            return 0

        lax.fori_loop(0, zrows, zloop, 0)
        pltpu.sync_copy(zv, shared.at[pl.ds(s * zrows, zrows)])

        for cp in copies:
            cp.wait()

        def wsloop(i, _):
            off = i * _L
            g16 = giv[pl.ds(off, _L)]
            cm16 = plsc.load_gather(cmv, [g16])
            rm16 = rmv[pl.ds(off, _L)]
            w16 = jnp.exp(rm16 - cm16)
            for j in range(_L):
                wb = jnp.full((_L,), w16[j], jnp.float32)
                r = off + j
                for k2 in range(_D // _L):
                    qv[r, pl.ds(k2 * _L, _L)] = qv[r, pl.ds(k2 * _L, _L)] * wb
            return 0

        lax.fori_loop(0, _RPT // _L, wsloop, 0)

        plsc.subcore_barrier()
        scats = [
            pltpu.async_copy(qv.at[pl.ds(j * 128, 128)],
                             shared.at[giv2.at[j]], sem, add=True)
            for j in range(_RPT // 128)
        ]
        for cp in scats:
            cp.wait()
        plsc.subcore_barrier()
        pltpu.sync_copy(shared.at[pl.ds(s * zrows, zrows)],
                        qu_hbm.at[c, pl.ds(s * zrows, zrows)])

    return k(q_rows, rowmax, gidx, colmax)


def _tc_c(q_t, mem, qu, br):
    def body(q_ref, mem_ref, qu_ref, w1_ref, w2_ref, uq_ref, mem2_ref,
             m2s, m2ts):
        b = pl.program_id(0)
        s = pl.program_id(1)

        @pl.when(jnp.logical_and(b == 0, s == 0))
        def _():
            m = mem_ref[...] + qu_ref[0]
            nrm = jnp.sqrt(jnp.sum(m * m, axis=1, keepdims=True))
            m2 = m / jnp.maximum(nrm, 1e-12)
            m2s[...] = m2
            mem2_ref[...] = m2
            m2ts[...] = lax.dot_general(_eye(_D), m2, (((1,), (1,)), ((), ())),
                                        preferred_element_type=jnp.float32)

        m2 = m2s[...]
        q = q_ref[0]
        st = lax.dot_general(m2, q, (((1,), (0,)), ((), ())),
                             preferred_element_type=jnp.float32)
        e = jnp.exp(st)
        den = jnp.sum(e, axis=0, keepdims=True)
        att2 = lax.dot_general(m2ts[...], e, (((1,), (0,)), ((), ())),
                               preferred_element_type=jnp.float32)
        att2 = att2 / den
        f1 = lax.dot_general(w1_ref[...], q, (((1,), (0,)), ((), ())),
                             preferred_element_type=jnp.float32)
        f2 = lax.dot_general(w2_ref[...], att2, (((1,), (0,)), ((), ())),
                             preferred_element_type=jnp.float32)
        num = jnp.sum(f1 * f2, axis=0)
        den2 = jnp.sqrt(jnp.sum(f1 * f1, axis=0)) * jnp.sqrt(jnp.sum(f2 * f2, axis=0))
        sim = num / jnp.maximum(den2, 1e-8)
        uq_ref[0] = q * sim[None, :]

    f32 = jnp.float32

    def call(w1, w2):
        return pl.pallas_call(
            body,
            grid=(2, _NS),
            in_specs=[
                pl.BlockSpec((1, _D, _C), lambda b, s: (b, 0, s)),
                pl.BlockSpec((_M, _D), lambda b, s: (0, 0)),
                pl.BlockSpec((1, _M, _D), lambda b, s: (br, 0, 0)),
                pl.BlockSpec((_D, _D), lambda b, s: (0, 0)),
                pl.BlockSpec((_D, _D), lambda b, s: (0, 0)),
            ],
            out_specs=[
                pl.BlockSpec((1, _D, _C), lambda b, s: (b, 0, s)),
                pl.BlockSpec((_M, _D), lambda b, s: (0, 0)),
            ],
            out_shape=[
                jax.ShapeDtypeStruct((2, _D, 4096), f32),
                jax.ShapeDtypeStruct((_M, _D), f32),
            ],
            scratch_shapes=[pltpu.VMEM((_M, _D), f32),
                            pltpu.VMEM((_D, _M), f32)],
        )(q_t, mem, qu, w1, w2)

    return call


def kernel(feat, tar, real_mem, fake_mem, theta1_w, theta2_w):
    del tar
    b, d, h, w = feat.shape
    feat_c = feat.reshape(b, d, h * w)
    (q_t, q_rows, rm, gi, cm) = _tc_a(feat_c, real_mem, fake_mem)
    qu = _sc_b(q_rows, rm.reshape(2, _N), gi.reshape(2, _N),
               cm.reshape(2, _M))
    uq_r, mem2_r = _tc_c(q_t, real_mem, qu, 0)(theta1_w, theta2_w)
    uq_f, mem2_f = _tc_c(q_t, fake_mem, qu, 1)(theta1_w, theta2_w)
    feat_out = q_t.reshape(b, d, h, w)
    return (uq_r.reshape(b, d, h, w), feat_out, mem2_r,
            uq_f.reshape(b, d, h, w), feat_out, mem2_f)


# speedup vs baseline: 4.6968x; 1.0797x over previous
import functools

import jax
import jax.numpy as jnp
from jax import lax
from jax.experimental import pallas as pl
from jax.experimental.pallas import tpu as pltpu
from jax.experimental.pallas import tpu_sc as plsc

def _eye(n):
    r = lax.broadcasted_iota(jnp.int32, (n, n), 0)
    c = lax.broadcasted_iota(jnp.int32, (n, n), 1)
    return (r == c).astype(jnp.float32)


_N = 8192
_M = 2048
_D = 128
_C = 1024
_NS = 4096 // _C
_RPT = _N // 16
_L = 16


def _tc_a(feat_c, real_mem, fake_mem):
    def body(feat_ref, rmem_ref, fmem_ref, qt_ref, qr_ref, rm_ref,
             gi_ref, cm_ref, mt):
        b = pl.program_id(0)
        s = pl.program_id(1)

        @pl.when(jnp.logical_and(b == 0, s == 0))
        def _():
            eye = _eye(_D)
            mt[:, :_M] = lax.dot_general(eye, rmem_ref[...],
                                         (((1,), (1,)), ((), ())),
                                         preferred_element_type=jnp.float32)
            mt[:, _M:] = lax.dot_general(eye, fmem_ref[...],
                                         (((1,), (1,)), ((), ())),
                                         preferred_element_type=jnp.float32)

        f = feat_ref[0]
        nrm = jnp.sqrt(jnp.sum(f * f, axis=0, keepdims=True))
        q = f / jnp.maximum(nrm, 1e-12)
        qt_ref[0] = q
        qr = lax.dot_general(q, _eye(_D), (((0,), (0,)), ((), ())),
                             preferred_element_type=jnp.float32)
        qr_ref[...] = qr
        st = lax.dot_general(qr, mt[...], (((1,), (0,)), ((), ())),
                             preferred_element_type=jnp.float32)
        for br in range(2):
            sb = st[:, br * _M:(br + 1) * _M]
            rmax = jnp.max(sb, axis=1)
            gi = jnp.argmax(sb, axis=1).astype(jnp.int32)
            rm_ref[br, 0, :, 0] = rmax
            gi_ref[br, 0, :, 0] = gi
            cmt = jnp.max(sb, axis=0)

            @pl.when(jnp.logical_and(b == 0, s == 0))
            def _():
                cm_ref[br, 0, :] = cmt

            @pl.when(jnp.logical_or(b != 0, s != 0))
            def _():
                cm_ref[br, 0, :] = jnp.maximum(cm_ref[br, 0, :], cmt)

    f32 = jnp.float32
    i32 = jnp.int32
    nt = 2 * _NS
    return pl.pallas_call(
        body,
        grid=(2, _NS),
        in_specs=[
            pl.BlockSpec((1, _D, _C), lambda b, s: (b, 0, s)),
            pl.BlockSpec((_M, _D), lambda b, s: (0, 0)),
            pl.BlockSpec((_M, _D), lambda b, s: (0, 0)),
        ],
        out_specs=[
            pl.BlockSpec((1, _D, _C), lambda b, s: (b, 0, s)),
            pl.BlockSpec((_C, _D), lambda b, s: (b * _NS + s, 0)),
            pl.BlockSpec((2, 1, _C, 1), lambda b, s: (0, b * _NS + s, 0, 0)),
            pl.BlockSpec((2, 1, _C, 1), lambda b, s: (0, b * _NS + s, 0, 0)),
            pl.BlockSpec((2, 1, _M), lambda b, s: (0, 0, 0)),
        ],
        out_shape=[
            jax.ShapeDtypeStruct((2, _D, 4096), f32),
            jax.ShapeDtypeStruct((_N, _D), f32),
            jax.ShapeDtypeStruct((2, nt, _C, 1), f32),
            jax.ShapeDtypeStruct((2, nt, _C, 1), i32),
            jax.ShapeDtypeStruct((2, 1, _M), f32),
        ],
        scratch_shapes=[pltpu.VMEM((_D, 2 * _M), f32)],
    )(feat_c, real_mem, fake_mem)


def _sc_b(q_rows, rowmax, gidx, colmax):
    mesh = plsc.VectorSubcoreMesh(core_axis_name="c", subcore_axis_name="s")

    @functools.partial(
        pl.kernel,
        out_type=jax.ShapeDtypeStruct((2, _M, _D), jnp.float32),
        mesh=mesh,
        compiler_params=pltpu.CompilerParams(needs_layout_passes=False),
        scratch_types=[
            pltpu.VMEM((_RPT, _D), jnp.float32),
            pltpu.VMEM((_RPT,), jnp.float32),
            pltpu.VMEM((_RPT,), jnp.int32),
            pltpu.VMEM((_RPT // 128, 128), jnp.int32),
            pltpu.VMEM((_M,), jnp.float32),
            pltpu.VMEM((_M // 16, _D), jnp.float32),
            pltpu.VMEM_SHARED((_M, _D), jnp.float32),
            pltpu.SemaphoreType.DMA,
        ],
    )
    def k(q_hbm, rm_hbm, gi_hbm, cm_hbm, qu_hbm,
          qv, rmv, giv, giv2, cmv, zv, shared, sem):
        c = lax.axis_index("c")
        s = lax.axis_index("s")
        base = s * _RPT

        copies = [
            pltpu.async_copy(q_hbm.at[pl.ds(base, _RPT)], qv, sem),
            pltpu.async_copy(rm_hbm.at[c, pl.ds(base, _RPT)], rmv, sem),
            pltpu.async_copy(gi_hbm.at[c, pl.ds(base, _RPT)], giv, sem),
            pltpu.async_copy(cm_hbm.at[c], cmv, sem),
        ] + [
            pltpu.async_copy(gi_hbm.at[c, pl.ds(base + j * 128, 128)],
                             giv2.at[j], sem)
            for j in range(_RPT // 128)
        ]
        zero16 = jnp.zeros((_L,), jnp.float32)
        zrows = _M // 16

        def zloop(i, _):
            for k2 in range(_D // _L):
                zv[i, pl.ds(k2 * _L, _L)] = zero16
            return 0

        lax.fori_loop(0, zrows, zloop, 0)
        pltpu.sync_copy(zv, shared.at[pl.ds(s * zrows, zrows)])

        for cp in copies:
            cp.wait()

        def wsloop(i, _):
            off = i * _L
            g16 = giv[pl.ds(off, _L)]
            cm16 = plsc.load_gather(cmv, [g16])
            rm16 = rmv[pl.ds(off, _L)]
            w16 = jnp.exp(rm16 - cm16)
            for j in range(_L):
                wb = jnp.full((_L,), w16[j], jnp.float32)
                r = off + j
                for k2 in range(_D // _L):
                    qv[r, pl.ds(k2 * _L, _L)] = qv[r, pl.ds(k2 * _L, _L)] * wb
            return 0

        lax.fori_loop(0, _RPT // _L, wsloop, 0)

        plsc.subcore_barrier()
        scats = [
            pltpu.async_copy(qv.at[pl.ds(j * 128, 128)],
                             shared.at[giv2.at[j]], sem, add=True)
            for j in range(_RPT // 128)
        ]
        for cp in scats:
            cp.wait()
        plsc.subcore_barrier()
        pltpu.sync_copy(shared.at[pl.ds(s * zrows, zrows)],
                        qu_hbm.at[c, pl.ds(s * zrows, zrows)])

    return k(q_rows, rowmax, gidx, colmax)


def _tc_c(q_t, mem, qu, br):
    def body(q_ref, mem_ref, qu_ref, w1_ref, w2_ref, uq_ref, mem2_ref,
             m2s, m2ts):
        b = pl.program_id(0)
        s = pl.program_id(1)

        @pl.when(jnp.logical_and(b == 0, s == 0))
        def _():
            m = mem_ref[...] + qu_ref[0]
            nrm = jnp.sqrt(jnp.sum(m * m, axis=1, keepdims=True))
            m2 = m / jnp.maximum(nrm, 1e-12)
            m2s[...] = m2
            mem2_ref[...] = m2
            m2ts[...] = lax.dot_general(_eye(_D), m2, (((1,), (1,)), ((), ())),
                                        preferred_element_type=jnp.float32)

        m2 = m2s[...]
        q = q_ref[0]
        st = lax.dot_general(m2, q, (((1,), (0,)), ((), ())),
                             preferred_element_type=jnp.float32)
        e = jnp.exp(st)
        den = jnp.sum(e, axis=0, keepdims=True)
        att2 = lax.dot_general(m2ts[...], e, (((1,), (0,)), ((), ())),
                               preferred_element_type=jnp.float32)
        att2 = att2 / den
        f1 = lax.dot_general(w1_ref[...], q, (((1,), (0,)), ((), ())),
                             preferred_element_type=jnp.float32)
        f2 = lax.dot_general(w2_ref[...], att2, (((1,), (0,)), ((), ())),
                             preferred_element_type=jnp.float32)
        num = jnp.sum(f1 * f2, axis=0)
        den2 = jnp.sqrt(jnp.sum(f1 * f1, axis=0)) * jnp.sqrt(jnp.sum(f2 * f2, axis=0))
        sim = num / jnp.maximum(den2, 1e-8)
        uq_ref[0] = q * sim[None, :]

    f32 = jnp.float32

    def call(w1, w2):
        return pl.pallas_call(
            body,
            grid=(2, _NS),
            in_specs=[
                pl.BlockSpec((1, _D, _C), lambda b, s: (b, 0, s)),
                pl.BlockSpec((_M, _D), lambda b, s: (0, 0)),
                pl.BlockSpec((1, _M, _D), lambda b, s: (br, 0, 0)),
                pl.BlockSpec((_D, _D), lambda b, s: (0, 0)),
                pl.BlockSpec((_D, _D), lambda b, s: (0, 0)),
            ],
            out_specs=[
                pl.BlockSpec((1, _D, _C), lambda b, s: (b, 0, s)),
                pl.BlockSpec((_M, _D), lambda b, s: (0, 0)),
            ],
            out_shape=[
                jax.ShapeDtypeStruct((2, _D, 4096), f32),
                jax.ShapeDtypeStruct((_M, _D), f32),
            ],
            scratch_shapes=[pltpu.VMEM((_M, _D), f32),
                            pltpu.VMEM((_D, _M), f32)],
        )(q_t, mem, qu, w1, w2)

    return call


def kernel(feat, tar, real_mem, fake_mem, theta1_w, theta2_w):
    del tar
    b, d, h, w = feat.shape
    feat_c = feat.reshape(b, d, h * w)
    (q_t, q_rows, rm, gi, cm) = _tc_a(feat_c, real_mem, fake_mem)
    qu = _sc_b(q_rows, rm.reshape(2, _N), gi.reshape(2, _N),
               cm.reshape(2, _M))
    uq_r, mem2_r = _tc_c(q_t, real_mem, qu, 0)(theta1_w, theta2_w)
    uq_f, mem2_f = _tc_c(q_t, fake_mem, qu, 1)(theta1_w, theta2_w)
    feat_out = q_t.reshape(b, d, h, w)
    return (uq_r.reshape(b, d, h, w), feat_out, mem2_r,
            uq_f.reshape(b, d, h, w), feat_out, mem2_f)


# speedup vs baseline: 4.7364x; 1.0084x over previous
import functools

import jax
import jax.numpy as jnp
from jax import lax
from jax.experimental import pallas as pl
from jax.experimental.pallas import tpu as pltpu
from jax.experimental.pallas import tpu_sc as plsc

def _eye(n):
    r = lax.broadcasted_iota(jnp.int32, (n, n), 0)
    c = lax.broadcasted_iota(jnp.int32, (n, n), 1)
    return (r == c).astype(jnp.float32)


_N = 8192
_M = 2048
_D = 128
_C = 2048
_NS = 4096 // _C
_RPT = _N // 16
_L = 16


def _tc_a(feat_c, real_mem, fake_mem):
    def body(feat_ref, rmem_ref, fmem_ref, qt_ref, qr_ref, rm_ref,
             gi_ref, cm_ref, mt):
        b = pl.program_id(0)
        s = pl.program_id(1)

        @pl.when(jnp.logical_and(b == 0, s == 0))
        def _():
            eye = _eye(_D)
            mt[:, :_M] = lax.dot_general(eye, rmem_ref[...],
                                         (((1,), (1,)), ((), ())),
                                         preferred_element_type=jnp.float32)
            mt[:, _M:] = lax.dot_general(eye, fmem_ref[...],
                                         (((1,), (1,)), ((), ())),
                                         preferred_element_type=jnp.float32)

        f = feat_ref[0]
        nrm = jnp.sqrt(jnp.sum(f * f, axis=0, keepdims=True))
        q = f / jnp.maximum(nrm, 1e-12)
        qt_ref[0] = q
        qr = lax.dot_general(q, _eye(_D), (((0,), (0,)), ((), ())),
                             preferred_element_type=jnp.float32)
        qr_ref[...] = qr
        st = lax.dot_general(qr, mt[...], (((1,), (0,)), ((), ())),
                             preferred_element_type=jnp.float32)
        for br in range(2):
            sb = st[:, br * _M:(br + 1) * _M]
            rmax = jnp.max(sb, axis=1)
            gi = jnp.argmax(sb, axis=1).astype(jnp.int32)
            rm_ref[br, 0, :, 0] = rmax
            gi_ref[br, 0, :, 0] = gi
            cmt = jnp.max(sb, axis=0)

            @pl.when(jnp.logical_and(b == 0, s == 0))
            def _():
                cm_ref[br, 0, :] = cmt

            @pl.when(jnp.logical_or(b != 0, s != 0))
            def _():
                cm_ref[br, 0, :] = jnp.maximum(cm_ref[br, 0, :], cmt)

    f32 = jnp.float32
    i32 = jnp.int32
    nt = 2 * _NS
    return pl.pallas_call(
        body,
        grid=(2, _NS),
        in_specs=[
            pl.BlockSpec((1, _D, _C), lambda b, s: (b, 0, s)),
            pl.BlockSpec((_M, _D), lambda b, s: (0, 0)),
            pl.BlockSpec((_M, _D), lambda b, s: (0, 0)),
        ],
        out_specs=[
            pl.BlockSpec((1, _D, _C), lambda b, s: (b, 0, s)),
            pl.BlockSpec((_C, _D), lambda b, s: (b * _NS + s, 0)),
            pl.BlockSpec((2, 1, _C, 1), lambda b, s: (0, b * _NS + s, 0, 0)),
            pl.BlockSpec((2, 1, _C, 1), lambda b, s: (0, b * _NS + s, 0, 0)),
            pl.BlockSpec((2, 1, _M), lambda b, s: (0, 0, 0)),
        ],
        out_shape=[
            jax.ShapeDtypeStruct((2, _D, 4096), f32),
            jax.ShapeDtypeStruct((_N, _D), f32),
            jax.ShapeDtypeStruct((2, nt, _C, 1), f32),
            jax.ShapeDtypeStruct((2, nt, _C, 1), i32),
            jax.ShapeDtypeStruct((2, 1, _M), f32),
        ],
        scratch_shapes=[pltpu.VMEM((_D, 2 * _M), f32)],
    )(feat_c, real_mem, fake_mem)


def _sc_b(q_rows, rowmax, gidx, colmax):
    mesh = plsc.VectorSubcoreMesh(core_axis_name="c", subcore_axis_name="s")

    @functools.partial(
        pl.kernel,
        out_type=jax.ShapeDtypeStruct((2, _M, _D), jnp.float32),
        mesh=mesh,
        compiler_params=pltpu.CompilerParams(needs_layout_passes=False),
        scratch_types=[
            pltpu.VMEM((_RPT, _D), jnp.float32),
            pltpu.VMEM((_RPT,), jnp.float32),
            pltpu.VMEM((_RPT,), jnp.int32),
            pltpu.VMEM((_RPT // 128, 128), jnp.int32),
            pltpu.VMEM((_M,), jnp.float32),
            pltpu.VMEM((_M // 16, _D), jnp.float32),
            pltpu.VMEM_SHARED((_M, _D), jnp.float32),
            pltpu.SemaphoreType.DMA,
        ],
    )
    def k(q_hbm, rm_hbm, gi_hbm, cm_hbm, qu_hbm,
          qv, rmv, giv, giv2, cmv, zv, shared, sem):
        c = lax.axis_index("c")
        s = lax.axis_index("s")
        base = s * _RPT

        copies = [
            pltpu.async_copy(q_hbm.at[pl.ds(base, _RPT)], qv, sem),
            pltpu.async_copy(rm_hbm.at[c, pl.ds(base, _RPT)], rmv, sem),
            pltpu.async_copy(gi_hbm.at[c, pl.ds(base, _RPT)], giv, sem),
            pltpu.async_copy(cm_hbm.at[c], cmv, sem),
        ] + [
            pltpu.async_copy(gi_hbm.at[c, pl.ds(base + j * 128, 128)],
                             giv2.at[j], sem)
            for j in range(_RPT // 128)
        ]
        zero16 = jnp.zeros((_L,), jnp.float32)
        zrows = _M // 16

        def zloop(i, _):
            for k2 in range(_D // _L):
                zv[i, pl.ds(k2 * _L, _L)] = zero16
            return 0

        lax.fori_loop(0, zrows, zloop, 0)
        pltpu.sync_copy(zv, shared.at[pl.ds(s * zrows, zrows)])

        for cp in copies:
            cp.wait()

        def wsloop(i, _):
            off = i * _L
            g16 = giv[pl.ds(off, _L)]
            cm16 = plsc.load_gather(cmv, [g16])
            rm16 = rmv[pl.ds(off, _L)]
            w16 = jnp.exp(rm16 - cm16)
            for j in range(_L):
                wb = jnp.full((_L,), w16[j], jnp.float32)
                r = off + j
                for k2 in range(_D // _L):
                    qv[r, pl.ds(k2 * _L, _L)] = qv[r, pl.ds(k2 * _L, _L)] * wb
            return 0

        lax.fori_loop(0, _RPT // _L, wsloop, 0)

        plsc.subcore_barrier()
        scats = [
            pltpu.async_copy(qv.at[pl.ds(j * 128, 128)],
                             shared.at[giv2.at[j]], sem, add=True)
            for j in range(_RPT // 128)
        ]
        for cp in scats:
            cp.wait()
        plsc.subcore_barrier()
        pltpu.sync_copy(shared.at[pl.ds(s * zrows, zrows)],
                        qu_hbm.at[c, pl.ds(s * zrows, zrows)])

    return k(q_rows, rowmax, gidx, colmax)


def _tc_c(q_t, mem, qu, br):
    def body(q_ref, mem_ref, qu_ref, w1_ref, w2_ref, uq_ref, mem2_ref,
             m2s, m2ts):
        b = pl.program_id(0)
        s = pl.program_id(1)

        @pl.when(jnp.logical_and(b == 0, s == 0))
        def _():
            m = mem_ref[...] + qu_ref[0]
            nrm = jnp.sqrt(jnp.sum(m * m, axis=1, keepdims=True))
            m2 = m / jnp.maximum(nrm, 1e-12)
            m2s[...] = m2
            mem2_ref[...] = m2
            m2ts[...] = lax.dot_general(_eye(_D), m2, (((1,), (1,)), ((), ())),
                                        preferred_element_type=jnp.float32)

        m2 = m2s[...]
        q = q_ref[0]
        st = lax.dot_general(m2, q, (((1,), (0,)), ((), ())),
                             preferred_element_type=jnp.float32)
        e = jnp.exp(st)
        den = jnp.sum(e, axis=0, keepdims=True)
        att2 = lax.dot_general(m2ts[...], e, (((1,), (0,)), ((), ())),
                               preferred_element_type=jnp.float32)
        att2 = att2 / den
        f1 = lax.dot_general(w1_ref[...], q, (((1,), (0,)), ((), ())),
                             preferred_element_type=jnp.float32)
        f2 = lax.dot_general(w2_ref[...], att2, (((1,), (0,)), ((), ())),
                             preferred_element_type=jnp.float32)
        num = jnp.sum(f1 * f2, axis=0)
        den2 = jnp.sqrt(jnp.sum(f1 * f1, axis=0)) * jnp.sqrt(jnp.sum(f2 * f2, axis=0))
        sim = num / jnp.maximum(den2, 1e-8)
        uq_ref[0] = q * sim[None, :]

    f32 = jnp.float32

    def call(w1, w2):
        return pl.pallas_call(
            body,
            grid=(2, _NS),
            in_specs=[
                pl.BlockSpec((1, _D, _C), lambda b, s: (b, 0, s)),
                pl.BlockSpec((_M, _D), lambda b, s: (0, 0)),
                pl.BlockSpec((1, _M, _D), lambda b, s: (br, 0, 0)),
                pl.BlockSpec((_D, _D), lambda b, s: (0, 0)),
                pl.BlockSpec((_D, _D), lambda b, s: (0, 0)),
            ],
            out_specs=[
                pl.BlockSpec((1, _D, _C), lambda b, s: (b, 0, s)),
                pl.BlockSpec((_M, _D), lambda b, s: (0, 0)),
            ],
            out_shape=[
                jax.ShapeDtypeStruct((2, _D, 4096), f32),
                jax.ShapeDtypeStruct((_M, _D), f32),
            ],
            scratch_shapes=[pltpu.VMEM((_M, _D), f32),
                            pltpu.VMEM((_D, _M), f32)],
        )(q_t, mem, qu, w1, w2)

    return call


def kernel(feat, tar, real_mem, fake_mem, theta1_w, theta2_w):
    del tar
    b, d, h, w = feat.shape
    feat_c = feat.reshape(b, d, h * w)
    (q_t, q_rows, rm, gi, cm) = _tc_a(feat_c, real_mem, fake_mem)
    qu = _sc_b(q_rows, rm.reshape(2, _N), gi.reshape(2, _N),
               cm.reshape(2, _M))
    uq_r, mem2_r = _tc_c(q_t, real_mem, qu, 0)(theta1_w, theta2_w)
    uq_f, mem2_f = _tc_c(q_t, fake_mem, qu, 1)(theta1_w, theta2_w)
    feat_out = q_t.reshape(b, d, h, w)
    return (uq_r.reshape(b, d, h, w), feat_out, mem2_r,
            uq_f.reshape(b, d, h, w), feat_out, mem2_f)


# speedup vs baseline: 4.8775x; 1.0298x over previous
import functools

import jax
import jax.numpy as jnp
from jax import lax
from jax.experimental import pallas as pl
from jax.experimental.pallas import tpu as pltpu
from jax.experimental.pallas import tpu_sc as plsc

def _eye(n):
    r = lax.broadcasted_iota(jnp.int32, (n, n), 0)
    c = lax.broadcasted_iota(jnp.int32, (n, n), 1)
    return (r == c).astype(jnp.float32)


_N = 8192
_M = 2048
_D = 128
_C = 2048
_NS = 4096 // _C
_RPT = _N // 32
_L = 16


def _tc_a(feat_c, real_mem, fake_mem):
    def body(feat_ref, rmem_ref, fmem_ref, qt_ref, qr_ref, rm_ref,
             gi_ref, cm_ref, mt):
        b = pl.program_id(0)
        s = pl.program_id(1)

        @pl.when(jnp.logical_and(b == 0, s == 0))
        def _():
            eye = _eye(_D)
            mt[:, :_M] = lax.dot_general(eye, rmem_ref[...],
                                         (((1,), (1,)), ((), ())),
                                         preferred_element_type=jnp.float32)
            mt[:, _M:] = lax.dot_general(eye, fmem_ref[...],
                                         (((1,), (1,)), ((), ())),
                                         preferred_element_type=jnp.float32)

        f = feat_ref[0]
        nrm = jnp.sqrt(jnp.sum(f * f, axis=0, keepdims=True))
        q = f / jnp.maximum(nrm, 1e-12)
        qt_ref[0] = q
        qr = lax.dot_general(q, _eye(_D), (((0,), (0,)), ((), ())),
                             preferred_element_type=jnp.float32)
        qr_ref[...] = qr
        st = lax.dot_general(qr, mt[...], (((1,), (0,)), ((), ())),
                             preferred_element_type=jnp.float32)
        for br in range(2):
            sb = st[:, br * _M:(br + 1) * _M]
            rmax = jnp.max(sb, axis=1)
            gi = jnp.argmax(sb, axis=1).astype(jnp.int32)
            rm_ref[br, 0, :, 0] = rmax
            gi_ref[br, 0, :, 0] = gi
            cmt = jnp.max(sb, axis=0)

            @pl.when(jnp.logical_and(b == 0, s == 0))
            def _():
                cm_ref[br, 0, :] = cmt

            @pl.when(jnp.logical_or(b != 0, s != 0))
            def _():
                cm_ref[br, 0, :] = jnp.maximum(cm_ref[br, 0, :], cmt)

    f32 = jnp.float32
    i32 = jnp.int32
    nt = 2 * _NS
    return pl.pallas_call(
        body,
        grid=(2, _NS),
        in_specs=[
            pl.BlockSpec((1, _D, _C), lambda b, s: (b, 0, s)),
            pl.BlockSpec((_M, _D), lambda b, s: (0, 0)),
            pl.BlockSpec((_M, _D), lambda b, s: (0, 0)),
        ],
        out_specs=[
            pl.BlockSpec((1, _D, _C), lambda b, s: (b, 0, s)),
            pl.BlockSpec((_C, _D), lambda b, s: (b * _NS + s, 0)),
            pl.BlockSpec((2, 1, _C, 1), lambda b, s: (0, b * _NS + s, 0, 0)),
            pl.BlockSpec((2, 1, _C, 1), lambda b, s: (0, b * _NS + s, 0, 0)),
            pl.BlockSpec((2, 1, _M), lambda b, s: (0, 0, 0)),
        ],
        out_shape=[
            jax.ShapeDtypeStruct((2, _D, 4096), f32),
            jax.ShapeDtypeStruct((_N, _D), f32),
            jax.ShapeDtypeStruct((2, nt, _C, 1), f32),
            jax.ShapeDtypeStruct((2, nt, _C, 1), i32),
            jax.ShapeDtypeStruct((2, 1, _M), f32),
        ],
        scratch_shapes=[pltpu.VMEM((_D, 2 * _M), f32)],
    )(feat_c, real_mem, fake_mem)


def _sc_b(q_rows, rowmax, gidx, colmax, br):
    mesh = plsc.VectorSubcoreMesh(core_axis_name="c", subcore_axis_name="s")

    @functools.partial(
        pl.kernel,
        out_type=jax.ShapeDtypeStruct((2, _M, _D), jnp.float32),
        mesh=mesh,
        compiler_params=pltpu.CompilerParams(needs_layout_passes=False),
        scratch_types=[
            pltpu.VMEM((_RPT, _D), jnp.float32),
            pltpu.VMEM((_RPT,), jnp.float32),
            pltpu.VMEM((_RPT,), jnp.int32),
            pltpu.VMEM((_RPT // 128, 128), jnp.int32),
            pltpu.VMEM((_M,), jnp.float32),
            pltpu.VMEM((_M // 16, _D), jnp.float32),
            pltpu.VMEM_SHARED((_M, _D), jnp.float32),
            pltpu.SemaphoreType.DMA,
        ],
    )
    def k(q_hbm, rm_hbm, gi_hbm, cm_hbm, qu_hbm,
          qv, rmv, giv, giv2, cmv, zv, shared, sem):
        c = lax.axis_index("c")
        s = lax.axis_index("s")
        base = (s * 2 + c) * _RPT

        copies = [
            pltpu.async_copy(q_hbm.at[pl.ds(base, _RPT)], qv, sem),
            pltpu.async_copy(rm_hbm.at[br, pl.ds(base, _RPT)], rmv, sem),
            pltpu.async_copy(gi_hbm.at[br, pl.ds(base, _RPT)], giv, sem),
            pltpu.async_copy(cm_hbm.at[br], cmv, sem),
        ] + [
            pltpu.async_copy(gi_hbm.at[br, pl.ds(base + j * 128, 128)],
                             giv2.at[j], sem)
            for j in range(_RPT // 128)
        ]
        zero16 = jnp.zeros((_L,), jnp.float32)
        zrows = _M // 16

        def zloop(i, _):
            for k2 in range(_D // _L):
                zv[i, pl.ds(k2 * _L, _L)] = zero16
            return 0

        lax.fori_loop(0, zrows, zloop, 0)
        pltpu.sync_copy(zv, shared.at[pl.ds(s * zrows, zrows)])

        for cp in copies:
            cp.wait()

        def wsloop(i, _):
            off = i * _L
            g16 = giv[pl.ds(off, _L)]
            cm16 = plsc.load_gather(cmv, [g16])
            rm16 = rmv[pl.ds(off, _L)]
            w16 = jnp.exp(rm16 - cm16)
            for j in range(_L):
                wb = jnp.full((_L,), w16[j], jnp.float32)
                r = off + j
                for k2 in range(_D // _L):
                    qv[r, pl.ds(k2 * _L, _L)] = qv[r, pl.ds(k2 * _L, _L)] * wb
            return 0

        lax.fori_loop(0, _RPT // _L, wsloop, 0)

        plsc.subcore_barrier()
        scats = [
            pltpu.async_copy(qv.at[pl.ds(j * 128, 128)],
                             shared.at[giv2.at[j]], sem, add=True)
            for j in range(_RPT // 128)
        ]
        for cp in scats:
            cp.wait()
        plsc.subcore_barrier()
        pltpu.sync_copy(shared.at[pl.ds(s * zrows, zrows)],
                        qu_hbm.at[c, pl.ds(s * zrows, zrows)])

    return k(q_rows, rowmax, gidx, colmax)


def _tc_c(q_t, mem, qu):
    def body(q_ref, mem_ref, qu_ref, w1_ref, w2_ref, uq_ref, mem2_ref,
             m2s, m2ts):
        b = pl.program_id(0)
        s = pl.program_id(1)

        @pl.when(jnp.logical_and(b == 0, s == 0))
        def _():
            m = mem_ref[...] + qu_ref[0] + qu_ref[1]
            nrm = jnp.sqrt(jnp.sum(m * m, axis=1, keepdims=True))
            m2 = m / jnp.maximum(nrm, 1e-12)
            m2s[...] = m2
            mem2_ref[...] = m2
            m2ts[...] = lax.dot_general(_eye(_D), m2, (((1,), (1,)), ((), ())),
                                        preferred_element_type=jnp.float32)

        m2 = m2s[...]
        q = q_ref[0]
        st = lax.dot_general(m2, q, (((1,), (0,)), ((), ())),
                             preferred_element_type=jnp.float32)
        e = jnp.exp(st)
        den = jnp.sum(e, axis=0, keepdims=True)
        att2 = lax.dot_general(m2ts[...], e, (((1,), (0,)), ((), ())),
                               preferred_element_type=jnp.float32)
        att2 = att2 / den
        f1 = lax.dot_general(w1_ref[...], q, (((1,), (0,)), ((), ())),
                             preferred_element_type=jnp.float32)
        f2 = lax.dot_general(w2_ref[...], att2, (((1,), (0,)), ((), ())),
                             preferred_element_type=jnp.float32)
        num = jnp.sum(f1 * f2, axis=0)
        den2 = jnp.sqrt(jnp.sum(f1 * f1, axis=0)) * jnp.sqrt(jnp.sum(f2 * f2, axis=0))
        sim = num / jnp.maximum(den2, 1e-8)
        uq_ref[0] = q * sim[None, :]

    f32 = jnp.float32

    def call(w1, w2):
        return pl.pallas_call(
            body,
            grid=(2, _NS),
            in_specs=[
                pl.BlockSpec((1, _D, _C), lambda b, s: (b, 0, s)),
                pl.BlockSpec((_M, _D), lambda b, s: (0, 0)),
                pl.BlockSpec((2, _M, _D), lambda b, s: (0, 0, 0)),
                pl.BlockSpec((_D, _D), lambda b, s: (0, 0)),
                pl.BlockSpec((_D, _D), lambda b, s: (0, 0)),
            ],
            out_specs=[
                pl.BlockSpec((1, _D, _C), lambda b, s: (b, 0, s)),
                pl.BlockSpec((_M, _D), lambda b, s: (0, 0)),
            ],
            out_shape=[
                jax.ShapeDtypeStruct((2, _D, 4096), f32),
                jax.ShapeDtypeStruct((_M, _D), f32),
            ],
            scratch_shapes=[pltpu.VMEM((_M, _D), f32),
                            pltpu.VMEM((_D, _M), f32)],
        )(q_t, mem, qu, w1, w2)

    return call


def kernel(feat, tar, real_mem, fake_mem, theta1_w, theta2_w):
    del tar
    b, d, h, w = feat.shape
    feat_c = feat.reshape(b, d, h * w)
    (q_t, q_rows, rm, gi, cm) = _tc_a(feat_c, real_mem, fake_mem)
    rm2, gi2, cm2 = rm.reshape(2, _N), gi.reshape(2, _N), cm.reshape(2, _M)
    qu_r = _sc_b(q_rows, rm2, gi2, cm2, 0)
    qu_f = _sc_b(q_rows, rm2, gi2, cm2, 1)
    uq_r, mem2_r = _tc_c(q_t, real_mem, qu_r)(theta1_w, theta2_w)
    uq_f, mem2_f = _tc_c(q_t, fake_mem, qu_f)(theta1_w, theta2_w)
    feat_out = q_t.reshape(b, d, h, w)
    return (uq_r.reshape(b, d, h, w), feat_out, mem2_r,
            uq_f.reshape(b, d, h, w), feat_out, mem2_f)


# speedup vs baseline: 5.0968x; 1.0450x over previous
import functools

import jax
import jax.numpy as jnp
from jax import lax
from jax.experimental import pallas as pl
from jax.experimental.pallas import tpu as pltpu
from jax.experimental.pallas import tpu_sc as plsc

def _eye(n):
    r = lax.broadcasted_iota(jnp.int32, (n, n), 0)
    c = lax.broadcasted_iota(jnp.int32, (n, n), 1)
    return (r == c).astype(jnp.float32)


_N = 8192
_M = 2048
_D = 128
_C = 2048
_NS = 4096 // _C
_RPT = _N // 16
_L = 16


def _tc_a(feat_c, real_mem, fake_mem):
    def body(feat_ref, rmem_ref, fmem_ref, qt_ref, qr_ref, rm_ref,
             gi_ref, cm_ref, mt):
        b = pl.program_id(0)
        s = pl.program_id(1)

        @pl.when(jnp.logical_and(b == 0, s == 0))
        def _():
            eye = _eye(_D)
            mt[:, :_M] = lax.dot_general(eye, rmem_ref[...],
                                         (((1,), (1,)), ((), ())),
                                         preferred_element_type=jnp.float32)
            mt[:, _M:] = lax.dot_general(eye, fmem_ref[...],
                                         (((1,), (1,)), ((), ())),
                                         preferred_element_type=jnp.float32)

        f = feat_ref[0]
        nrm = jnp.sqrt(jnp.sum(f * f, axis=0, keepdims=True))
        q = f / jnp.maximum(nrm, 1e-12)
        qt_ref[0] = q
        qr = lax.dot_general(q, _eye(_D), (((0,), (0,)), ((), ())),
                             preferred_element_type=jnp.float32)
        qr_ref[...] = qr
        st = lax.dot_general(qr, mt[...], (((1,), (0,)), ((), ())),
                             preferred_element_type=jnp.float32)
        for br in range(2):
            sb = st[:, br * _M:(br + 1) * _M]
            rmax = jnp.max(sb, axis=1)
            gi = jnp.argmax(sb, axis=1).astype(jnp.int32)
            rm_ref[br, 0] = rmax.reshape(_C // 128, 128)
            gi_ref[br, 0] = gi.reshape(_C // 128, 128)
            cmt = jnp.max(sb, axis=0)

            @pl.when(jnp.logical_and(b == 0, s == 0))
            def _():
                cm_ref[br, 0, :] = cmt

            @pl.when(jnp.logical_or(b != 0, s != 0))
            def _():
                cm_ref[br, 0, :] = jnp.maximum(cm_ref[br, 0, :], cmt)

    f32 = jnp.float32
    i32 = jnp.int32
    nt = 2 * _NS
    return pl.pallas_call(
        body,
        grid=(2, _NS),
        in_specs=[
            pl.BlockSpec((1, _D, _C), lambda b, s: (b, 0, s)),
            pl.BlockSpec((_M, _D), lambda b, s: (0, 0)),
            pl.BlockSpec((_M, _D), lambda b, s: (0, 0)),
        ],
        out_specs=[
            pl.BlockSpec((1, _D, _C), lambda b, s: (b, 0, s)),
            pl.BlockSpec((_C, _D), lambda b, s: (b * _NS + s, 0)),
            pl.BlockSpec((2, 1, _C // 128, 128),
                         lambda b, s: (0, b * _NS + s, 0, 0)),
            pl.BlockSpec((2, 1, _C // 128, 128),
                         lambda b, s: (0, b * _NS + s, 0, 0)),
            pl.BlockSpec((2, 1, _M), lambda b, s: (0, 0, 0)),
        ],
        out_shape=[
            jax.ShapeDtypeStruct((2, _D, 4096), f32),
            jax.ShapeDtypeStruct((_N, _D), f32),
            jax.ShapeDtypeStruct((2, nt, _C // 128, 128), f32),
            jax.ShapeDtypeStruct((2, nt, _C // 128, 128), i32),
            jax.ShapeDtypeStruct((2, 1, _M), f32),
        ],
        scratch_shapes=[pltpu.VMEM((_D, 2 * _M), f32)],
    )(feat_c, real_mem, fake_mem)


def _sc_b(q_rows, rowmax, gidx, colmax):
    mesh = plsc.VectorSubcoreMesh(core_axis_name="c", subcore_axis_name="s")

    @functools.partial(
        pl.kernel,
        out_type=jax.ShapeDtypeStruct((2, _M, _D), jnp.float32),
        mesh=mesh,
        compiler_params=pltpu.CompilerParams(needs_layout_passes=False),
        scratch_types=[
            pltpu.VMEM((_RPT, _D), jnp.float32),
            pltpu.VMEM((_RPT,), jnp.float32),
            pltpu.VMEM((_RPT,), jnp.int32),
            pltpu.VMEM((_RPT // 128, 128), jnp.int32),
            pltpu.VMEM((_M,), jnp.float32),
            pltpu.VMEM((_M // 16, _D), jnp.float32),
            pltpu.VMEM_SHARED((_M, _D), jnp.float32),
            pltpu.SemaphoreType.DMA,
        ],
    )
    def k(q_hbm, rm_hbm, gi_hbm, cm_hbm, qu_hbm,
          qv, rmv, giv, giv2, cmv, zv, shared, sem):
        c = lax.axis_index("c")
        s = lax.axis_index("s")
        base = s * _RPT

        copies = [
            pltpu.async_copy(q_hbm.at[pl.ds(base, _RPT)], qv, sem),
            pltpu.async_copy(rm_hbm.at[c, pl.ds(base, _RPT)], rmv, sem),
            pltpu.async_copy(gi_hbm.at[c, pl.ds(base, _RPT)], giv, sem),
            pltpu.async_copy(cm_hbm.at[c], cmv, sem),
        ] + [
            pltpu.async_copy(gi_hbm.at[c, pl.ds(base + j * 128, 128)],
                             giv2.at[j], sem)
            for j in range(_RPT // 128)
        ]
        zero16 = jnp.zeros((_L,), jnp.float32)
        zrows = _M // 16

        def zloop(i, _):
            for k2 in range(_D // _L):
                zv[i, pl.ds(k2 * _L, _L)] = zero16
            return 0

        lax.fori_loop(0, zrows, zloop, 0)
        pltpu.sync_copy(zv, shared.at[pl.ds(s * zrows, zrows)])

        for cp in copies:
            cp.wait()

        def wsloop(i, _):
            off = i * _L
            g16 = giv[pl.ds(off, _L)]
            cm16 = plsc.load_gather(cmv, [g16])
            rm16 = rmv[pl.ds(off, _L)]
            w16 = jnp.exp(rm16 - cm16)
            for j in range(_L):
                wb = jnp.full((_L,), w16[j], jnp.float32)
                r = off + j
                for k2 in range(_D // _L):
                    qv[r, pl.ds(k2 * _L, _L)] = qv[r, pl.ds(k2 * _L, _L)] * wb
            return 0

        lax.fori_loop(0, _RPT // _L, wsloop, 0)

        plsc.subcore_barrier()
        scats = [
            pltpu.async_copy(qv.at[pl.ds(j * 128, 128)],
                             shared.at[giv2.at[j]], sem, add=True)
            for j in range(_RPT // 128)
        ]
        for cp in scats:
            cp.wait()
        plsc.subcore_barrier()
        pltpu.sync_copy(shared.at[pl.ds(s * zrows, zrows)],
                        qu_hbm.at[c, pl.ds(s * zrows, zrows)])

    return k(q_rows, rowmax, gidx, colmax)


def _tc_c(q_t, mem, qu, br):
    def body(q_ref, mem_ref, qu_ref, w1_ref, w2_ref, uq_ref, mem2_ref,
             m2s, m2ts):
        b = pl.program_id(0)
        s = pl.program_id(1)

        @pl.when(jnp.logical_and(b == 0, s == 0))
        def _():
            m = mem_ref[...] + qu_ref[0]
            nrm = jnp.sqrt(jnp.sum(m * m, axis=1, keepdims=True))
            m2 = m / jnp.maximum(nrm, 1e-12)
            m2s[...] = m2
            mem2_ref[...] = m2
            m2ts[...] = lax.dot_general(_eye(_D), m2, (((1,), (1,)), ((), ())),
                                        preferred_element_type=jnp.float32)

        m2 = m2s[...]
        q = q_ref[0]
        st = lax.dot_general(m2, q, (((1,), (0,)), ((), ())),
                             preferred_element_type=jnp.float32)
        e = jnp.exp(st)
        den = jnp.sum(e, axis=0, keepdims=True)
        att2 = lax.dot_general(m2ts[...], e, (((1,), (0,)), ((), ())),
                               preferred_element_type=jnp.float32)
        att2 = att2 / den
        f1 = lax.dot_general(w1_ref[...], q, (((1,), (0,)), ((), ())),
                             preferred_element_type=jnp.float32)
        f2 = lax.dot_general(w2_ref[...], att2, (((1,), (0,)), ((), ())),
                             preferred_element_type=jnp.float32)
        num = jnp.sum(f1 * f2, axis=0)
        den2 = jnp.sqrt(jnp.sum(f1 * f1, axis=0)) * jnp.sqrt(jnp.sum(f2 * f2, axis=0))
        sim = num / jnp.maximum(den2, 1e-8)
        uq_ref[0] = q * sim[None, :]

    f32 = jnp.float32

    def call(w1, w2):
        return pl.pallas_call(
            body,
            grid=(2, _NS),
            in_specs=[
                pl.BlockSpec((1, _D, _C), lambda b, s: (b, 0, s)),
                pl.BlockSpec((_M, _D), lambda b, s: (0, 0)),
                pl.BlockSpec((1, _M, _D), lambda b, s: (br, 0, 0)),
                pl.BlockSpec((_D, _D), lambda b, s: (0, 0)),
                pl.BlockSpec((_D, _D), lambda b, s: (0, 0)),
            ],
            out_specs=[
                pl.BlockSpec((1, _D, _C), lambda b, s: (b, 0, s)),
                pl.BlockSpec((_M, _D), lambda b, s: (0, 0)),
            ],
            out_shape=[
                jax.ShapeDtypeStruct((2, _D, 4096), f32),
                jax.ShapeDtypeStruct((_M, _D), f32),
            ],
            scratch_shapes=[pltpu.VMEM((_M, _D), f32),
                            pltpu.VMEM((_D, _M), f32)],
        )(q_t, mem, qu, w1, w2)

    return call


def kernel(feat, tar, real_mem, fake_mem, theta1_w, theta2_w):
    del tar
    b, d, h, w = feat.shape
    feat_c = feat.reshape(b, d, h * w)
    (q_t, q_rows, rm, gi, cm) = _tc_a(feat_c, real_mem, fake_mem)
    rm2, gi2, cm2 = rm.reshape(2, _N), gi.reshape(2, _N), cm.reshape(2, _M)
    qu = _sc_b(q_rows, rm2, gi2, cm2)
    uq_r, mem2_r = _tc_c(q_t, real_mem, qu, 0)(theta1_w, theta2_w)
    uq_f, mem2_f = _tc_c(q_t, fake_mem, qu, 1)(theta1_w, theta2_w)
    feat_out = q_t.reshape(b, d, h, w)
    return (uq_r.reshape(b, d, h, w), feat_out, mem2_r,
            uq_f.reshape(b, d, h, w), feat_out, mem2_f)


# speedup vs baseline: 5.2197x; 1.0241x over previous
import functools

import jax
import jax.numpy as jnp
from jax import lax
from jax.experimental import pallas as pl
from jax.experimental.pallas import tpu as pltpu
from jax.experimental.pallas import tpu_sc as plsc

def _eye(n):
    r = lax.broadcasted_iota(jnp.int32, (n, n), 0)
    c = lax.broadcasted_iota(jnp.int32, (n, n), 1)
    return (r == c).astype(jnp.float32)


_N = 8192
_M = 2048
_D = 128
_C = 2048
_NS = 4096 // _C
_RPT = _N // 32
_L = 16


def _tc_a(feat_c, real_mem, fake_mem):
    def body(feat_ref, rmem_ref, fmem_ref, qt_ref, qr_ref, rm_ref,
             gi_ref, cm_ref, mt):
        b = pl.program_id(0)
        s = pl.program_id(1)

        @pl.when(jnp.logical_and(b == 0, s == 0))
        def _():
            eye = _eye(_D)
            mt[:, :_M] = lax.dot_general(eye, rmem_ref[...],
                                         (((1,), (1,)), ((), ())),
                                         preferred_element_type=jnp.float32)
            mt[:, _M:] = lax.dot_general(eye, fmem_ref[...],
                                         (((1,), (1,)), ((), ())),
                                         preferred_element_type=jnp.float32)

        f = feat_ref[0]
        nrm = jnp.sqrt(jnp.sum(f * f, axis=0, keepdims=True))
        q = f / jnp.maximum(nrm, 1e-12)
        qt_ref[0] = q
        qr = lax.dot_general(q, _eye(_D), (((0,), (0,)), ((), ())),
                             preferred_element_type=jnp.float32)
        qr_ref[...] = qr
        st = lax.dot_general(qr, mt[...], (((1,), (0,)), ((), ())),
                             preferred_element_type=jnp.float32)
        for br in range(2):
            sb = st[:, br * _M:(br + 1) * _M]
            rmax = jnp.max(sb, axis=1)
            gi = jnp.argmax(sb, axis=1).astype(jnp.int32)
            rm_ref[br, 0] = rmax.reshape(_C // 128, 128)
            gi_ref[br, 0] = gi.reshape(_C // 128, 128)
            cmt = jnp.max(sb, axis=0)

            @pl.when(jnp.logical_and(b == 0, s == 0))
            def _():
                cm_ref[br, 0, :] = cmt

            @pl.when(jnp.logical_or(b != 0, s != 0))
            def _():
                cm_ref[br, 0, :] = jnp.maximum(cm_ref[br, 0, :], cmt)

    f32 = jnp.float32
    i32 = jnp.int32
    nt = 2 * _NS
    return pl.pallas_call(
        body,
        grid=(2, _NS),
        in_specs=[
            pl.BlockSpec((1, _D, _C), lambda b, s: (b, 0, s)),
            pl.BlockSpec((_M, _D), lambda b, s: (0, 0)),
            pl.BlockSpec((_M, _D), lambda b, s: (0, 0)),
        ],
        out_specs=[
            pl.BlockSpec((1, _D, _C), lambda b, s: (b, 0, s)),
            pl.BlockSpec((_C, _D), lambda b, s: (b * _NS + s, 0)),
            pl.BlockSpec((2, 1, _C // 128, 128),
                         lambda b, s: (0, b * _NS + s, 0, 0)),
            pl.BlockSpec((2, 1, _C // 128, 128),
                         lambda b, s: (0, b * _NS + s, 0, 0)),
            pl.BlockSpec((2, 1, _M), lambda b, s: (0, 0, 0)),
        ],
        out_shape=[
            jax.ShapeDtypeStruct((2, _D, 4096), f32),
            jax.ShapeDtypeStruct((_N, _D), f32),
            jax.ShapeDtypeStruct((2, nt, _C // 128, 128), f32),
            jax.ShapeDtypeStruct((2, nt, _C // 128, 128), i32),
            jax.ShapeDtypeStruct((2, 1, _M), f32),
        ],
        scratch_shapes=[pltpu.VMEM((_D, 2 * _M), f32)],
    )(feat_c, real_mem, fake_mem)


def _sc_b(q_rows, rowmax, gidx, colmax, br):
    mesh = plsc.VectorSubcoreMesh(core_axis_name="c", subcore_axis_name="s")

    @functools.partial(
        pl.kernel,
        out_type=jax.ShapeDtypeStruct((2, _M, _D), jnp.float32),
        mesh=mesh,
        compiler_params=pltpu.CompilerParams(needs_layout_passes=False),
        scratch_types=[
            pltpu.VMEM((_RPT, _D), jnp.float32),
            pltpu.VMEM((_RPT,), jnp.float32),
            pltpu.VMEM((_RPT,), jnp.int32),
            pltpu.VMEM((_RPT // 128, 128), jnp.int32),
            pltpu.VMEM((_M,), jnp.float32),
            pltpu.VMEM((_M // 16, _D), jnp.float32),
            pltpu.VMEM_SHARED((_M, _D), jnp.float32),
            pltpu.SemaphoreType.DMA,
        ],
    )
    def k(q_hbm, rm_hbm, gi_hbm, cm_hbm, qu_hbm,
          qv, rmv, giv, giv2, cmv, zv, shared, sem):
        c = lax.axis_index("c")
        s = lax.axis_index("s")
        base = (s * 2 + c) * _RPT

        copies = [
            pltpu.async_copy(q_hbm.at[pl.ds(base, _RPT)], qv, sem),
            pltpu.async_copy(rm_hbm.at[br, pl.ds(base, _RPT)], rmv, sem),
            pltpu.async_copy(gi_hbm.at[br, pl.ds(base, _RPT)], giv, sem),
            pltpu.async_copy(cm_hbm.at[br], cmv, sem),
        ] + [
            pltpu.async_copy(gi_hbm.at[br, pl.ds(base + j * 128, 128)],
                             giv2.at[j], sem)
            for j in range(_RPT // 128)
        ]
        zero16 = jnp.zeros((_L,), jnp.float32)
        zrows = _M // 16

        def zloop(i, _):
            for k2 in range(_D // _L):
                zv[i, pl.ds(k2 * _L, _L)] = zero16
            return 0

        lax.fori_loop(0, zrows, zloop, 0)
        pltpu.sync_copy(zv, shared.at[pl.ds(s * zrows, zrows)])

        for cp in copies:
            cp.wait()

        def wsloop(i, _):
            off = i * _L
            g16 = giv[pl.ds(off, _L)]
            cm16 = plsc.load_gather(cmv, [g16])
            rm16 = rmv[pl.ds(off, _L)]
            w16 = jnp.exp(rm16 - cm16)
            for j in range(_L):
                wb = jnp.full((_L,), w16[j], jnp.float32)
                r = off + j
                for k2 in range(_D // _L):
                    qv[r, pl.ds(k2 * _L, _L)] = qv[r, pl.ds(k2 * _L, _L)] * wb
            return 0

        lax.fori_loop(0, _RPT // _L, wsloop, 0)

        plsc.subcore_barrier()
        scats = [
            pltpu.async_copy(qv.at[pl.ds(j * 128, 128)],
                             shared.at[giv2.at[j]], sem, add=True)
            for j in range(_RPT // 128)
        ]
        for cp in scats:
            cp.wait()
        plsc.subcore_barrier()
        pltpu.sync_copy(shared.at[pl.ds(s * zrows, zrows)],
                        qu_hbm.at[c, pl.ds(s * zrows, zrows)])

    return k(q_rows, rowmax, gidx, colmax)


def _tc_c(q_t, mem, qu):
    def body(q_ref, mem_ref, qu_ref, w1_ref, w2_ref, uq_ref, mem2_ref,
             m2s, m2ts):
        b = pl.program_id(0)
        s = pl.program_id(1)

        @pl.when(jnp.logical_and(b == 0, s == 0))
        def _():
            m = mem_ref[...] + qu_ref[0] + qu_ref[1]
            nrm = jnp.sqrt(jnp.sum(m * m, axis=1, keepdims=True))
            m2 = m / jnp.maximum(nrm, 1e-12)
            m2s[...] = m2
            mem2_ref[...] = m2
            m2ts[...] = lax.dot_general(_eye(_D), m2, (((1,), (1,)), ((), ())),
                                        preferred_element_type=jnp.float32)

        m2 = m2s[...]
        q = q_ref[0]
        st = lax.dot_general(m2, q, (((1,), (0,)), ((), ())),
                             preferred_element_type=jnp.float32)
        e = jnp.exp(st)
        den = jnp.sum(e, axis=0, keepdims=True)
        att2 = lax.dot_general(m2ts[...], e, (((1,), (0,)), ((), ())),
                               preferred_element_type=jnp.float32)
        att2 = att2 / den
        f1 = lax.dot_general(w1_ref[...], q, (((1,), (0,)), ((), ())),
                             preferred_element_type=jnp.float32)
        f2 = lax.dot_general(w2_ref[...], att2, (((1,), (0,)), ((), ())),
                             preferred_element_type=jnp.float32)
        num = jnp.sum(f1 * f2, axis=0)
        den2 = jnp.sqrt(jnp.sum(f1 * f1, axis=0)) * jnp.sqrt(jnp.sum(f2 * f2, axis=0))
        sim = num / jnp.maximum(den2, 1e-8)
        uq_ref[0] = q * sim[None, :]

    f32 = jnp.float32

    def call(w1, w2):
        return pl.pallas_call(
            body,
            grid=(2, _NS),
            in_specs=[
                pl.BlockSpec((1, _D, _C), lambda b, s: (b, 0, s)),
                pl.BlockSpec((_M, _D), lambda b, s: (0, 0)),
                pl.BlockSpec((2, _M, _D), lambda b, s: (0, 0, 0)),
                pl.BlockSpec((_D, _D), lambda b, s: (0, 0)),
                pl.BlockSpec((_D, _D), lambda b, s: (0, 0)),
            ],
            out_specs=[
                pl.BlockSpec((1, _D, _C), lambda b, s: (b, 0, s)),
                pl.BlockSpec((_M, _D), lambda b, s: (0, 0)),
            ],
            out_shape=[
                jax.ShapeDtypeStruct((2, _D, 4096), f32),
                jax.ShapeDtypeStruct((_M, _D), f32),
            ],
            scratch_shapes=[pltpu.VMEM((_M, _D), f32),
                            pltpu.VMEM((_D, _M), f32)],
        )(q_t, mem, qu, w1, w2)

    return call


def kernel(feat, tar, real_mem, fake_mem, theta1_w, theta2_w):
    del tar
    b, d, h, w = feat.shape
    feat_c = feat.reshape(b, d, h * w)
    (q_t, q_rows, rm, gi, cm) = _tc_a(feat_c, real_mem, fake_mem)
    rm2, gi2, cm2 = rm.reshape(2, _N), gi.reshape(2, _N), cm.reshape(2, _M)
    qu_r = _sc_b(q_rows, rm2, gi2, cm2, 0)
    qu_f = _sc_b(q_rows, rm2, gi2, cm2, 1)
    uq_r, mem2_r = _tc_c(q_t, real_mem, qu_r)(theta1_w, theta2_w)
    uq_f, mem2_f = _tc_c(q_t, fake_mem, qu_f)(theta1_w, theta2_w)
    feat_out = q_t.reshape(b, d, h, w)
    return (uq_r.reshape(b, d, h, w), feat_out, mem2_r,
            uq_f.reshape(b, d, h, w), feat_out, mem2_f)
